# baseline (speedup 1.0000x reference)
from functools import partial

import numpy as np
import jax
import jax.numpy as jnp
from jax import lax
from jax.experimental import pallas as pl
from jax.experimental.pallas import tpu as pltpu

_EPS = 1e-5
_INV_SQRT2 = 0.7071067811865475


def _gelu_exact(v):
    return 0.5 * v * (1.0 + lax.erf(v * _INV_SQRT2))


def _combine_dx(u, b, pp):
    return (u[1] + pltpu.roll(u[2], pp - 1, axis=1)
            + pltpu.roll(u[0], 1, axis=1) + b)


def _zero_margins(slab, rows, m, pp):
    z = jnp.zeros((rows, m), jnp.bfloat16)
    slab[:, 0:m] = z
    slab[:, m + pp:m + pp + m] = z


def _fill_rows(slab, val, nrows, width, stride, m, row0, rstep):
    zg = jnp.zeros((val.shape[0], stride - width), jnp.bfloat16)
    for i in range(nrows):
        r = row0 + i * rstep
        slab[:, m + i * stride:m + i * stride + width] = (
            val[:, r * width:(r + 1) * width])
        slab[:, m + i * stride + width:m + (i + 1) * stride] = zg


def _stage1(x_ref, skip_ref, muph_ref, wupc_ref, wupm_ref, wupp_ref,
            ws0_ref, ws1_ref, wsm_ref, wsp_ref, b1_ref, wsx_ref, wss_ref,
            bs_ref, mv_ref, y1_ref, s1_ref, q1_ref, ys_ref,
            up_ref, s0_ref, s1s_ref, *, stride, margin, pp, hh, width):
    c2, p4 = x_ref.shape[1], x_ref.shape[2]
    d = skip_ref.shape[1]
    m = margin
    i2 = pl.program_id(0) % 2
    up = up_ref.at[i2]
    s0 = s0_ref.at[i2]
    s1 = s1s_ref.at[i2]

    _zero_margins(up, c2, m, pp)
    _zero_margins(s0, d, m, pp)
    _zero_margins(s1, d, m, pp)

    uph = jnp.dot(x_ref[...].reshape(c2, p4).astype(jnp.bfloat16),
                  muph_ref[...], preferred_element_type=jnp.float32)
    up[:, m:m + pp] = uph.astype(jnp.bfloat16)

    sk = skip_ref[...].reshape(d, 2 * hh * width).astype(jnp.bfloat16)
    _fill_rows(s0, sk, hh, width, stride, m, 0, 2)
    _fill_rows(s1, sk, hh, width, stride, m, 1, 2)

    up_c = up[:, m:m + pp]
    up_m = up[:, m - stride:m - stride + pp]
    up_p = up[:, m + stride:m + stride + pp]
    s0_c = s0[:, m:m + pp]
    s0_p = s0[:, m + stride:m + stride + pp]
    s1_c = s1[:, m:m + pp]
    s1_m = s1[:, m - stride:m - stride + pp]

    u0 = [None, None, None]
    u1 = [None, None, None]
    for j in range(3):
        t = jnp.dot(wupc_ref[j], up_c, preferred_element_type=jnp.float32)
        u0[j], u1[j] = t[:d], t[d:]
        u0[j] += jnp.dot(wupm_ref[j], up_m, preferred_element_type=jnp.float32)
        u1[j] += jnp.dot(wupp_ref[j], up_p, preferred_element_type=jnp.float32)
        t = jnp.dot(ws0_ref[j], s0_c, preferred_element_type=jnp.float32)
        u0[j] += t[:d]
        u1[j] += t[d:]
        t = jnp.dot(ws1_ref[j], s1_c, preferred_element_type=jnp.float32)
        u0[j] += t[:d]
        u1[j] += t[d:]
        u0[j] += jnp.dot(wsm_ref[j], s1_m, preferred_element_type=jnp.float32)
        u1[j] += jnp.dot(wsp_ref[j], s0_p, preferred_element_type=jnp.float32)

    b1 = b1_ref[...]
    raw0 = _combine_dx(u0, b1, pp)
    raw1 = _combine_dx(u1, b1, pp)

    t = jnp.dot(wsx_ref[...], up_c, preferred_element_type=jnp.float32)
    bsv = bs_ref[...]
    ys0 = t + jnp.dot(wss_ref[...], s0_c,
                      preferred_element_type=jnp.float32) + bsv
    ys1 = t + jnp.dot(wss_ref[...], s1_c,
                      preferred_element_type=jnp.float32) + bsv

    y1_ref[0, 0] = raw0.astype(y1_ref.dtype)
    y1_ref[0, 1] = raw1.astype(y1_ref.dtype)
    ys_ref[0, 0] = ys0.astype(ys_ref.dtype)
    ys_ref[0, 1] = ys1.astype(ys_ref.dtype)
    mv = mv_ref[...]
    s1_ref[...] = (jnp.dot(raw0, mv, preferred_element_type=jnp.float32)
                   + jnp.dot(raw1, mv, preferred_element_type=jnp.float32)
                   ).reshape(1, d, 1)
    q1_ref[...] = (jnp.dot(raw0 * raw0, mv, preferred_element_type=jnp.float32)
                   + jnp.dot(raw1 * raw1, mv, preferred_element_type=jnp.float32)
                   ).reshape(1, d, 1)


def _stage2(y1_ref, sc1_ref, sh1_ref, gm_ref, wa0_ref, wa1_ref, wam_ref,
            wap_ref, b2_ref, mv_ref, y2_ref, s2_ref, q2_ref,
            a0_ref, a1s_ref, *, stride, margin, pp):
    d = y1_ref.shape[2]
    m = margin
    i2 = pl.program_id(0) % 2
    a0 = a0_ref.at[i2]
    a1 = a1s_ref.at[i2]

    _zero_margins(a0, d, m, pp)
    _zero_margins(a1, d, m, pp)

    sc, sh, gm = sc1_ref[...], sh1_ref[...], gm_ref[...]
    act0 = _gelu_exact(y1_ref[0, 0].astype(jnp.float32) * sc + sh)
    a0[:, m:m + pp] = act0.astype(jnp.bfloat16) * gm
    act1 = _gelu_exact(y1_ref[0, 1].astype(jnp.float32) * sc + sh)
    a1[:, m:m + pp] = act1.astype(jnp.bfloat16) * gm

    a0_c = a0[:, m:m + pp]
    a0_p = a0[:, m + stride:m + stride + pp]
    a1_c = a1[:, m:m + pp]
    a1_m = a1[:, m - stride:m - stride + pp]

    u0 = [None, None, None]
    u1 = [None, None, None]
    for j in range(3):
        t = jnp.dot(wa0_ref[j], a0_c, preferred_element_type=jnp.float32)
        u0[j], u1[j] = t[:d], t[d:]
        t = jnp.dot(wa1_ref[j], a1_c, preferred_element_type=jnp.float32)
        u0[j] += t[:d]
        u1[j] += t[d:]
        u0[j] += jnp.dot(wam_ref[j], a1_m, preferred_element_type=jnp.float32)
        u1[j] += jnp.dot(wap_ref[j], a0_p, preferred_element_type=jnp.float32)

    b2 = b2_ref[...]
    raw0 = _combine_dx(u0, b2, pp)
    raw1 = _combine_dx(u1, b2, pp)

    y2_ref[0, 0] = raw0.astype(y2_ref.dtype)
    y2_ref[0, 1] = raw1.astype(y2_ref.dtype)
    mv = mv_ref[...]
    s2_ref[...] = (jnp.dot(raw0, mv, preferred_element_type=jnp.float32)
                   + jnp.dot(raw1, mv, preferred_element_type=jnp.float32)
                   ).reshape(1, d, 1)
    q2_ref[...] = (jnp.dot(raw0 * raw0, mv, preferred_element_type=jnp.float32)
                   + jnp.dot(raw1 * raw1, mv, preferred_element_type=jnp.float32)
                   ).reshape(1, d, 1)


def _stage3(y2_ref, sc2_ref, sh2_ref, ys_ref, out_ref,
            *, stride, pp, hh, width):
    d = y2_ref.shape[2]
    sc, sh = sc2_ref[...], sh2_ref[...]
    for py in range(2):
        act = _gelu_exact(y2_ref[0, py].astype(jnp.float32) * sc + sh)
        v = act + ys_ref[0, py].astype(jnp.float32)
        for i in range(hh):
            r = 2 * i + py
            out_ref[0, :, r * width:(r + 1) * width] = (
                v[:, i * stride:i * stride + width])


def _finalize_bn(s, q, gamma, beta, count):
    tot = jnp.sum(s[:, :, 0], axis=0)
    totsq = jnp.sum(q[:, :, 0], axis=0)
    mu = tot / count
    var = totsq / count - mu * mu
    inv = lax.rsqrt(jnp.maximum(var, 0.0) + _EPS)
    sc = gamma * inv
    sh = beta - mu * sc
    d = sc.shape[0]
    return sc.reshape(d, 1), sh.reshape(d, 1)


def _params(sems):
    return pltpu.CompilerParams(dimension_semantics=sems,
                                vmem_limit_bytes=56 * 1024 * 1024)


def kernel(x, skip, w1, b1, g1, be1, w2, b2, g2, be2, wsx, wss, bs):
    n, c2, hh, ww = x.shape
    _, d, hgt, wid = skip.shape
    p4, p = hh * ww, hgt * wid
    stride = wid + 2
    pp = hh * stride
    m = max(128, pl.cdiv(stride + 1, 128) * 128)
    slen = 2 * m + pp
    bf16, f32 = jnp.bfloat16, jnp.float32

    xf = x.reshape(n, c2, p4)
    sf = skip.reshape(n, d, p)

    wu = w1[:, :, :c2]
    wsk = w1[:, :, c2:]

    def taps(w, dy):
        return jnp.stack([w[(dy + 1) * 3 + (dx + 1)] for dx in (-1, 0, 1)])

    wupc = jnp.concatenate([taps(wu, 0) + taps(wu, 1),
                            taps(wu, -1) + taps(wu, 0)], axis=1).astype(bf16)
    wupm = taps(wu, -1).astype(bf16)
    wupp = taps(wu, 1).astype(bf16)
    ws0 = jnp.concatenate([taps(wsk, 0), taps(wsk, -1)], axis=1).astype(bf16)
    ws1 = jnp.concatenate([taps(wsk, 1), taps(wsk, 0)], axis=1).astype(bf16)
    wsm = taps(wsk, -1).astype(bf16)
    wsp = taps(wsk, 1).astype(bf16)
    wa0 = jnp.concatenate([taps(w2, 0), taps(w2, -1)], axis=1).astype(bf16)
    wa1 = jnp.concatenate([taps(w2, 1), taps(w2, 0)], axis=1).astype(bf16)
    wam = taps(w2, -1).astype(bf16)
    wap = taps(w2, 1).astype(bf16)

    wsxb = wsx.astype(bf16)
    wssb = wss.astype(bf16)

    rr = np.arange(pp) // stride
    cc = np.arange(pp) % stride
    interior = cc < wid
    src = np.where(interior, rr * ww + np.minimum(cc, wid - 1) // 2, -1)
    muph = jnp.asarray(np.arange(p4)[:, None] == src[None, :], bf16)
    maskv = jnp.asarray(interior[:, None], f32)
    gmask = jnp.asarray(interior[None, :], bf16)

    cnst = lambda i: (0, 0)
    cnst3 = lambda i: (0, 0, 0)

    y1, s1, q1, ys = pl.pallas_call(
        partial(_stage1, stride=stride, margin=m, pp=pp, hh=hh, width=wid),
        grid=(n,),
        in_specs=[
            pl.BlockSpec((1, c2, p4), lambda i: (i, 0, 0)),
            pl.BlockSpec((1, d, p), lambda i: (i, 0, 0)),
            pl.BlockSpec((p4, pp), cnst),
            pl.BlockSpec((3, 2 * d, c2), cnst3),
            pl.BlockSpec((3, d, c2), cnst3),
            pl.BlockSpec((3, d, c2), cnst3),
            pl.BlockSpec((3, 2 * d, d), cnst3),
            pl.BlockSpec((3, 2 * d, d), cnst3),
            pl.BlockSpec((3, d, d), cnst3),
            pl.BlockSpec((3, d, d), cnst3),
            pl.BlockSpec((d, 1), cnst),
            pl.BlockSpec((d, c2), cnst),
            pl.BlockSpec((d, d), cnst),
            pl.BlockSpec((d, 1), cnst),
            pl.BlockSpec((pp, 1), cnst),
        ],
        out_specs=(
            pl.BlockSpec((1, 2, d, pp), lambda i: (i, 0, 0, 0)),
            pl.BlockSpec((1, d, 1), lambda i: (i, 0, 0)),
            pl.BlockSpec((1, d, 1), lambda i: (i, 0, 0)),
            pl.BlockSpec((1, 2, d, pp), lambda i: (i, 0, 0, 0)),
        ),
        out_shape=(
            jax.ShapeDtypeStruct((n, 2, d, pp), bf16),
            jax.ShapeDtypeStruct((n, d, 1), f32),
            jax.ShapeDtypeStruct((n, d, 1), f32),
            jax.ShapeDtypeStruct((n, 2, d, pp), bf16),
        ),
        scratch_shapes=[pltpu.VMEM((2, c2, slen), bf16),
                        pltpu.VMEM((2, d, slen), bf16),
                        pltpu.VMEM((2, d, slen), bf16)],
        compiler_params=_params(("parallel",)),
    )(xf, sf, muph, wupc, wupm, wupp, ws0, ws1, wsm, wsp, b1,
      wsxb, wssb, bs, maskv)

    sc1, sh1 = _finalize_bn(s1, q1, g1, be1, float(n * p))

    y2, s2, q2 = pl.pallas_call(
        partial(_stage2, stride=stride, margin=m, pp=pp),
        grid=(n,),
        in_specs=[
            pl.BlockSpec((1, 2, d, pp), lambda i: (i, 0, 0, 0)),
            pl.BlockSpec((d, 1), cnst),
            pl.BlockSpec((d, 1), cnst),
            pl.BlockSpec((1, pp), cnst),
            pl.BlockSpec((3, 2 * d, d), cnst3),
            pl.BlockSpec((3, 2 * d, d), cnst3),
            pl.BlockSpec((3, d, d), cnst3),
            pl.BlockSpec((3, d, d), cnst3),
            pl.BlockSpec((d, 1), cnst),
            pl.BlockSpec((pp, 1), cnst),
        ],
        out_specs=(
            pl.BlockSpec((1, 2, d, pp), lambda i: (i, 0, 0, 0)),
            pl.BlockSpec((1, d, 1), lambda i: (i, 0, 0)),
            pl.BlockSpec((1, d, 1), lambda i: (i, 0, 0)),
        ),
        out_shape=(
            jax.ShapeDtypeStruct((n, 2, d, pp), bf16),
            jax.ShapeDtypeStruct((n, d, 1), f32),
            jax.ShapeDtypeStruct((n, d, 1), f32),
        ),
        scratch_shapes=[pltpu.VMEM((2, d, slen), bf16),
                        pltpu.VMEM((2, d, slen), bf16)],
        compiler_params=_params(("parallel",)),
    )(y1, sc1, sh1, gmask, wa0, wa1, wam, wap, b2, maskv)

    sc2, sh2 = _finalize_bn(s2, q2, g2, be2, float(n * p))

    out = pl.pallas_call(
        partial(_stage3, stride=stride, pp=pp, hh=hh, width=wid),
        grid=(n,),
        in_specs=[
            pl.BlockSpec((1, 2, d, pp), lambda i: (i, 0, 0, 0)),
            pl.BlockSpec((d, 1), cnst),
            pl.BlockSpec((d, 1), cnst),
            pl.BlockSpec((1, 2, d, pp), lambda i: (i, 0, 0, 0)),
        ],
        out_specs=pl.BlockSpec((1, d, p), lambda i: (i, 0, 0)),
        out_shape=jax.ShapeDtypeStruct((n, d, p), f32),
        compiler_params=_params(("parallel",)),
    )(y2, sc2, sh2, ys)

    return out.reshape(n, d, hgt, wid)


# speedup vs baseline: 1.0165x; 1.0165x over previous
from functools import partial

import numpy as np
import jax
import jax.numpy as jnp
from jax import lax
from jax.experimental import pallas as pl
from jax.experimental.pallas import tpu as pltpu

_EPS = 1e-5
_INV_SQRT2 = 0.7071067811865475


def _gelu_exact(v):
    return 0.5 * v * (1.0 + lax.erf(v * _INV_SQRT2))


def _combine_dx(u, b, pp):
    return (u[1] + pltpu.roll(u[2], pp - 1, axis=1)
            + pltpu.roll(u[0], 1, axis=1) + b)


def _zero_margins(slab, rows, m, pp):
    z = jnp.zeros((rows, m), jnp.bfloat16)
    slab[:, 0:m] = z
    slab[:, m + pp:m + pp + m] = z


def _fill_rows(slab, val, nrows, width, stride, m, row0, rstep):
    zg = jnp.zeros((val.shape[0], stride - width), jnp.bfloat16)
    for i in range(nrows):
        r = row0 + i * rstep
        slab[:, m + i * stride:m + i * stride + width] = (
            val[:, r * width:(r + 1) * width])
        slab[:, m + i * stride + width:m + (i + 1) * stride] = zg


def _bn_from_sums(s_ref, q_ref, g_ref, be_ref, count):
    tot = s_ref[...]
    totsq = q_ref[...]
    mu = tot / count
    var = totsq / count - mu * mu
    inv = lax.rsqrt(jnp.maximum(var, 0.0) + _EPS)
    sc = g_ref[...] * inv
    sh = be_ref[...] - mu * sc
    return sc, sh


def _fused(x_ref, skip_ref, muph_ref, wupc_ref, wupm_ref, wupp_ref,
           ws0_ref, ws1_ref, wsm_ref, wsp_ref, b1_ref, wsx_ref, wss_ref,
           bs_ref, g1_ref, be1_ref, wa0_ref, wa1_ref, wam_ref, wap_ref,
           b2_ref, g2_ref, be2_ref, mv_ref, gm_ref,
           out_ref,
           up_ref, s0_ref, s1s_ref, a0_ref, a1s_ref,
           y1s_ref, yss_ref, y2s_ref,
           s1a_ref, q1a_ref, s2a_ref, q2a_ref,
           sc1_ref, sh1_ref, sc2_ref, sh2_ref,
           *, stride, margin, pp, hh, width, count):
    ph = pl.program_id(0)
    i = pl.program_id(1)
    c2, p4 = x_ref.shape[1], x_ref.shape[2]
    d = skip_ref.shape[1]
    m = margin
    mv = mv_ref[...]

    @pl.when(ph == 0)
    def _conv1():
        _zero_margins(up_ref, c2, m, pp)
        _zero_margins(s0_ref, d, m, pp)
        _zero_margins(s1s_ref, d, m, pp)

        uph = jnp.dot(x_ref[...].reshape(c2, p4).astype(jnp.bfloat16),
                      muph_ref[...], preferred_element_type=jnp.float32)
        up_ref[:, m:m + pp] = uph.astype(jnp.bfloat16)

        sk = skip_ref[...].reshape(d, 2 * hh * width).astype(jnp.bfloat16)
        _fill_rows(s0_ref, sk, hh, width, stride, m, 0, 2)
        _fill_rows(s1s_ref, sk, hh, width, stride, m, 1, 2)

        up_c = up_ref[:, m:m + pp]
        up_m = up_ref[:, m - stride:m - stride + pp]
        up_p = up_ref[:, m + stride:m + stride + pp]
        s0_c = s0_ref[:, m:m + pp]
        s0_p = s0_ref[:, m + stride:m + stride + pp]
        s1_c = s1s_ref[:, m:m + pp]
        s1_m = s1s_ref[:, m - stride:m - stride + pp]

        u0 = [None, None, None]
        u1 = [None, None, None]
        for j in range(3):
            t = jnp.dot(wupc_ref[j], up_c, preferred_element_type=jnp.float32)
            u0[j], u1[j] = t[:d], t[d:]
            u0[j] += jnp.dot(wupm_ref[j], up_m,
                             preferred_element_type=jnp.float32)
            u1[j] += jnp.dot(wupp_ref[j], up_p,
                             preferred_element_type=jnp.float32)
            t = jnp.dot(ws0_ref[j], s0_c, preferred_element_type=jnp.float32)
            u0[j] += t[:d]
            u1[j] += t[d:]
            t = jnp.dot(ws1_ref[j], s1_c, preferred_element_type=jnp.float32)
            u0[j] += t[:d]
            u1[j] += t[d:]
            u0[j] += jnp.dot(wsm_ref[j], s1_m,
                             preferred_element_type=jnp.float32)
            u1[j] += jnp.dot(wsp_ref[j], s0_p,
                             preferred_element_type=jnp.float32)

        b1 = b1_ref[...]
        raw0 = _combine_dx(u0, b1, pp)
        raw1 = _combine_dx(u1, b1, pp)

        t = jnp.dot(wsx_ref[...], up_c, preferred_element_type=jnp.float32)
        bsv = bs_ref[...]
        yss_ref[i, 0] = (t + jnp.dot(wss_ref[...], s0_c,
                                     preferred_element_type=jnp.float32)
                         + bsv).astype(jnp.bfloat16)
        yss_ref[i, 1] = (t + jnp.dot(wss_ref[...], s1_c,
                                     preferred_element_type=jnp.float32)
                         + bsv).astype(jnp.bfloat16)

        y1s_ref[i, 0] = raw0.astype(jnp.bfloat16)
        y1s_ref[i, 1] = raw1.astype(jnp.bfloat16)
        s = (jnp.dot(raw0, mv, preferred_element_type=jnp.float32)
             + jnp.dot(raw1, mv, preferred_element_type=jnp.float32))
        q = (jnp.dot(raw0 * raw0, mv, preferred_element_type=jnp.float32)
             + jnp.dot(raw1 * raw1, mv, preferred_element_type=jnp.float32))
        s1a_ref[...] = jnp.where(i == 0, s, s1a_ref[...] + s)
        q1a_ref[...] = jnp.where(i == 0, q, q1a_ref[...] + q)

    @pl.when((ph == 1) & (i == 0))
    def _bn1():
        sc, sh = _bn_from_sums(s1a_ref, q1a_ref, g1_ref, be1_ref, count)
        sc1_ref[...] = sc
        sh1_ref[...] = sh

    @pl.when(ph == 1)
    def _conv2():
        _zero_margins(a0_ref, d, m, pp)
        _zero_margins(a1s_ref, d, m, pp)

        sc, sh, gm = sc1_ref[...], sh1_ref[...], gm_ref[...]
        act0 = _gelu_exact(y1s_ref[i, 0].astype(jnp.float32) * sc + sh)
        a0_ref[:, m:m + pp] = act0.astype(jnp.bfloat16) * gm
        act1 = _gelu_exact(y1s_ref[i, 1].astype(jnp.float32) * sc + sh)
        a1s_ref[:, m:m + pp] = act1.astype(jnp.bfloat16) * gm

        a0_c = a0_ref[:, m:m + pp]
        a0_p = a0_ref[:, m + stride:m + stride + pp]
        a1_c = a1s_ref[:, m:m + pp]
        a1_m = a1s_ref[:, m - stride:m - stride + pp]

        u0 = [None, None, None]
        u1 = [None, None, None]
        for j in range(3):
            t = jnp.dot(wa0_ref[j], a0_c, preferred_element_type=jnp.float32)
            u0[j], u1[j] = t[:d], t[d:]
            t = jnp.dot(wa1_ref[j], a1_c, preferred_element_type=jnp.float32)
            u0[j] += t[:d]
            u1[j] += t[d:]
            u0[j] += jnp.dot(wam_ref[j], a1_m,
                             preferred_element_type=jnp.float32)
            u1[j] += jnp.dot(wap_ref[j], a0_p,
                             preferred_element_type=jnp.float32)

        b2 = b2_ref[...]
        raw0 = _combine_dx(u0, b2, pp)
        raw1 = _combine_dx(u1, b2, pp)

        y2s_ref[i, 0] = raw0.astype(jnp.bfloat16)
        y2s_ref[i, 1] = raw1.astype(jnp.bfloat16)
        s = (jnp.dot(raw0, mv, preferred_element_type=jnp.float32)
             + jnp.dot(raw1, mv, preferred_element_type=jnp.float32))
        q = (jnp.dot(raw0 * raw0, mv, preferred_element_type=jnp.float32)
             + jnp.dot(raw1 * raw1, mv, preferred_element_type=jnp.float32))
        s2a_ref[...] = jnp.where(i == 0, s, s2a_ref[...] + s)
        q2a_ref[...] = jnp.where(i == 0, q, q2a_ref[...] + q)

    @pl.when((ph == 2) & (i == 0))
    def _bn2():
        sc, sh = _bn_from_sums(s2a_ref, q2a_ref, g2_ref, be2_ref, count)
        sc2_ref[...] = sc
        sh2_ref[...] = sh

    @pl.when(ph == 2)
    def _epilogue():
        sc, sh = sc2_ref[...], sh2_ref[...]
        for py in range(2):
            act = _gelu_exact(y2s_ref[i, py].astype(jnp.float32) * sc + sh)
            v = act + yss_ref[i, py].astype(jnp.float32)
            for r in range(hh):
                fr = 2 * r + py
                out_ref[0, :, fr * width:(fr + 1) * width] = (
                    v[:, r * stride:r * stride + width])


def kernel(x, skip, w1, b1, g1, be1, w2, b2, g2, be2, wsx, wss, bs):
    n, c2, hh, ww = x.shape
    _, d, hgt, wid = skip.shape
    p4, p = hh * ww, hgt * wid
    stride = wid + 2
    pp = hh * stride
    m = max(128, pl.cdiv(stride + 1, 128) * 128)
    slen = 2 * m + pp
    bf16, f32 = jnp.bfloat16, jnp.float32

    xf = x.reshape(n, c2, p4)
    sf = skip.reshape(n, d, p)

    wu = w1[:, :, :c2]
    wsk = w1[:, :, c2:]

    def taps(w, dy):
        return jnp.stack([w[(dy + 1) * 3 + (dx + 1)] for dx in (-1, 0, 1)])

    wupc = jnp.concatenate([taps(wu, 0) + taps(wu, 1),
                            taps(wu, -1) + taps(wu, 0)], axis=1).astype(bf16)
    wupm = taps(wu, -1).astype(bf16)
    wupp = taps(wu, 1).astype(bf16)
    ws0 = jnp.concatenate([taps(wsk, 0), taps(wsk, -1)], axis=1).astype(bf16)
    ws1 = jnp.concatenate([taps(wsk, 1), taps(wsk, 0)], axis=1).astype(bf16)
    wsm = taps(wsk, -1).astype(bf16)
    wsp = taps(wsk, 1).astype(bf16)
    wa0 = jnp.concatenate([taps(w2, 0), taps(w2, -1)], axis=1).astype(bf16)
    wa1 = jnp.concatenate([taps(w2, 1), taps(w2, 0)], axis=1).astype(bf16)
    wam = taps(w2, -1).astype(bf16)
    wap = taps(w2, 1).astype(bf16)

    wsxb = wsx.astype(bf16)
    wssb = wss.astype(bf16)

    rr = np.arange(pp) // stride
    cc = np.arange(pp) % stride
    interior = cc < wid
    src = np.where(interior, rr * ww + np.minimum(cc, wid - 1) // 2, -1)
    muph = jnp.asarray(np.arange(p4)[:, None] == src[None, :], bf16)
    maskv = jnp.asarray(interior[:, None], f32)
    gmask = jnp.asarray(interior[None, :], bf16)

    g1c, be1c = g1.reshape(d, 1), be1.reshape(d, 1)
    g2c, be2c = g2.reshape(d, 1), be2.reshape(d, 1)

    def img0(pidx, ii):
        return (jnp.where(pidx == 0, ii, 0), 0, 0)

    cnst = lambda pidx, ii: (0, 0)
    cnst3 = lambda pidx, ii: (0, 0, 0)

    out = pl.pallas_call(
        partial(_fused, stride=stride, margin=m, pp=pp, hh=hh, width=wid,
                count=float(n * p)),
        grid=(3, n),
        in_specs=[
            pl.BlockSpec((1, c2, p4), img0),
            pl.BlockSpec((1, d, p), img0),
            pl.BlockSpec((p4, pp), cnst),
            pl.BlockSpec((3, 2 * d, c2), cnst3),
            pl.BlockSpec((3, d, c2), cnst3),
            pl.BlockSpec((3, d, c2), cnst3),
            pl.BlockSpec((3, 2 * d, d), cnst3),
            pl.BlockSpec((3, 2 * d, d), cnst3),
            pl.BlockSpec((3, d, d), cnst3),
            pl.BlockSpec((3, d, d), cnst3),
            pl.BlockSpec((d, 1), cnst),
            pl.BlockSpec((d, c2), cnst),
            pl.BlockSpec((d, d), cnst),
            pl.BlockSpec((d, 1), cnst),
            pl.BlockSpec((d, 1), cnst),
            pl.BlockSpec((d, 1), cnst),
            pl.BlockSpec((3, 2 * d, d), cnst3),
            pl.BlockSpec((3, 2 * d, d), cnst3),
            pl.BlockSpec((3, d, d), cnst3),
            pl.BlockSpec((3, d, d), cnst3),
            pl.BlockSpec((d, 1), cnst),
            pl.BlockSpec((d, 1), cnst),
            pl.BlockSpec((d, 1), cnst),
            pl.BlockSpec((pp, 1), cnst),
            pl.BlockSpec((1, pp), cnst),
        ],
        out_specs=pl.BlockSpec(
            (1, d, p), lambda pidx, ii: (jnp.where(pidx == 2, ii, 0), 0, 0)),
        out_shape=jax.ShapeDtypeStruct((n, d, p), f32),
        scratch_shapes=[
            pltpu.VMEM((c2, slen), bf16),
            pltpu.VMEM((d, slen), bf16),
            pltpu.VMEM((d, slen), bf16),
            pltpu.VMEM((d, slen), bf16),
            pltpu.VMEM((d, slen), bf16),
            pltpu.VMEM((n, 2, d, pp), bf16),
            pltpu.VMEM((n, 2, d, pp), bf16),
            pltpu.VMEM((n, 2, d, pp), bf16),
            pltpu.VMEM((d, 1), f32),
            pltpu.VMEM((d, 1), f32),
            pltpu.VMEM((d, 1), f32),
            pltpu.VMEM((d, 1), f32),
            pltpu.VMEM((d, 1), f32),
            pltpu.VMEM((d, 1), f32),
            pltpu.VMEM((d, 1), f32),
            pltpu.VMEM((d, 1), f32),
        ],
        compiler_params=pltpu.CompilerParams(
            dimension_semantics=("arbitrary", "arbitrary"),
            vmem_limit_bytes=56 * 1024 * 1024),
    )(xf, sf, muph, wupc, wupm, wupp, ws0, ws1, wsm, wsp, b1, wsxb, wssb,
      bs, g1c, be1c, wa0, wa1, wam, wap, b2, g2c, be2c, maskv, gmask)

    return out.reshape(n, d, hgt, wid)


# speedup vs baseline: 1.1378x; 1.1194x over previous
from functools import partial

import numpy as np
import jax
import jax.numpy as jnp
from jax import lax
from jax.experimental import pallas as pl
from jax.experimental.pallas import tpu as pltpu

_EPS = 1e-5
_INV_SQRT2 = 0.7071067811865475


def _gelu_exact(v):
    return 0.5 * v * (1.0 + lax.erf(v * _INV_SQRT2))


def _combine_dx(u, b, pp):
    return (u[1] + pltpu.roll(u[2], pp - 1, axis=1)
            + pltpu.roll(u[0], 1, axis=1) + b)


def _zero_margins(slab, rows, m, pp):
    z = jnp.zeros((rows, m), jnp.bfloat16)
    slab[:, 0:m] = z
    slab[:, m + pp:m + pp + m] = z


def _fill_rows(slab, r0, val, nrows, width, stride, m, row0, rstep):
    zg = jnp.zeros((val.shape[0], stride - width), jnp.bfloat16)
    for i in range(nrows):
        r = row0 + i * rstep
        slab[r0:r0 + val.shape[0],
             m + i * stride:m + i * stride + width] = (
            val[:, r * width:(r + 1) * width])
        slab[r0:r0 + val.shape[0],
             m + i * stride + width:m + (i + 1) * stride] = zg


def _bn_from_sums(s_ref, q_ref, g_ref, be_ref, count):
    tot = s_ref[...]
    totsq = q_ref[...]
    mu = tot / count
    var = totsq / count - mu * mu
    inv = lax.rsqrt(jnp.maximum(var, 0.0) + _EPS)
    sc = g_ref[...] * inv
    sh = be_ref[...] - mu * sc
    return sc, sh


def _conv9(mega, wc_ref, wm_ref, wp_ref, lo_rows, hi_rows, b, *,
           stride, m, pp, d):
    c_sl = mega[:, m:m + pp]
    m_sl = mega[0:lo_rows, m - stride:m - stride + pp]
    p_sl = mega[mega.shape[0] - hi_rows:, m + stride:m + stride + pp]
    u0 = [None, None, None]
    u1 = [None, None, None]
    for j in range(3):
        t = jnp.dot(wc_ref[j], c_sl, preferred_element_type=jnp.float32)
        u0[j] = t[:d] + jnp.dot(wm_ref[j], m_sl,
                                preferred_element_type=jnp.float32)
        u1[j] = t[d:] + jnp.dot(wp_ref[j], p_sl,
                                preferred_element_type=jnp.float32)
    return _combine_dx(u0, b, pp), _combine_dx(u1, b, pp)


def _fused(x_ref, skip_ref, muph_ref, wc1_ref, wm1_ref, wp1_ref, b1_ref,
           wys0_ref, wys1_ref, bs_ref, g1_ref, be1_ref,
           wc2_ref, wm2_ref, wp2_ref, b2_ref, g2_ref, be2_ref,
           mv_ref, gm_ref,
           out_ref,
           mega_ref, act_ref,
           y1s_ref, yss_ref, y2s_ref,
           s1a_ref, q1a_ref, s2a_ref, q2a_ref,
           sc1_ref, sh1_ref, sc2_ref, sh2_ref,
           *, stride, margin, pp, hh, width, count):
    ph = pl.program_id(0)
    i = pl.program_id(1)
    c2, p4 = x_ref.shape[1], x_ref.shape[2]
    d = skip_ref.shape[1]
    m = margin
    mv = mv_ref[...]

    @pl.when(ph == 0)
    def _conv1():
        _zero_margins(mega_ref, d + c2 + d, m, pp)

        uph = jnp.dot(x_ref[...].reshape(c2, p4).astype(jnp.bfloat16),
                      muph_ref[...], preferred_element_type=jnp.float32)
        mega_ref[d:d + c2, m:m + pp] = uph.astype(jnp.bfloat16)

        sk = skip_ref[...].reshape(d, 2 * hh * width).astype(jnp.bfloat16)
        _fill_rows(mega_ref, 0, sk, hh, width, stride, m, 1, 2)
        _fill_rows(mega_ref, d + c2, sk, hh, width, stride, m, 0, 2)

        raw0, raw1 = _conv9(mega_ref, wc1_ref, wm1_ref, wp1_ref,
                            d + c2, c2 + d, b1_ref[...],
                            stride=stride, m=m, pp=pp, d=d)

        bsv = bs_ref[...]
        yss_ref[i, 0] = (jnp.dot(wys0_ref[...], mega_ref[d:, m:m + pp],
                                 preferred_element_type=jnp.float32)
                         + bsv).astype(jnp.bfloat16)
        yss_ref[i, 1] = (jnp.dot(wys1_ref[...], mega_ref[:d + c2, m:m + pp],
                                 preferred_element_type=jnp.float32)
                         + bsv).astype(jnp.bfloat16)

        y1s_ref[i, 0] = raw0.astype(jnp.bfloat16)
        y1s_ref[i, 1] = raw1.astype(jnp.bfloat16)
        s = (jnp.dot(raw0, mv, preferred_element_type=jnp.float32)
             + jnp.dot(raw1, mv, preferred_element_type=jnp.float32))
        q = (jnp.dot(raw0 * raw0, mv, preferred_element_type=jnp.float32)
             + jnp.dot(raw1 * raw1, mv, preferred_element_type=jnp.float32))
        s1a_ref[...] = jnp.where(i == 0, s, s1a_ref[...] + s)
        q1a_ref[...] = jnp.where(i == 0, q, q1a_ref[...] + q)

    @pl.when((ph == 1) & (i == 0))
    def _bn1():
        sc, sh = _bn_from_sums(s1a_ref, q1a_ref, g1_ref, be1_ref, count)
        sc1_ref[...] = sc
        sh1_ref[...] = sh

    @pl.when(ph == 1)
    def _conv2():
        _zero_margins(act_ref, 2 * d, m, pp)

        sc, sh, gm = sc1_ref[...], sh1_ref[...], gm_ref[...]
        act1 = _gelu_exact(y1s_ref[i, 1].astype(jnp.float32) * sc + sh)
        act_ref[0:d, m:m + pp] = act1.astype(jnp.bfloat16) * gm
        act0 = _gelu_exact(y1s_ref[i, 0].astype(jnp.float32) * sc + sh)
        act_ref[d:, m:m + pp] = act0.astype(jnp.bfloat16) * gm

        raw0, raw1 = _conv9(act_ref, wc2_ref, wm2_ref, wp2_ref,
                            d, d, b2_ref[...],
                            stride=stride, m=m, pp=pp, d=d)

        y2s_ref[i, 0] = raw0.astype(jnp.bfloat16)
        y2s_ref[i, 1] = raw1.astype(jnp.bfloat16)
        s = (jnp.dot(raw0, mv, preferred_element_type=jnp.float32)
             + jnp.dot(raw1, mv, preferred_element_type=jnp.float32))
        q = (jnp.dot(raw0 * raw0, mv, preferred_element_type=jnp.float32)
             + jnp.dot(raw1 * raw1, mv, preferred_element_type=jnp.float32))
        s2a_ref[...] = jnp.where(i == 0, s, s2a_ref[...] + s)
        q2a_ref[...] = jnp.where(i == 0, q, q2a_ref[...] + q)

    @pl.when((ph == 2) & (i == 0))
    def _bn2():
        sc, sh = _bn_from_sums(s2a_ref, q2a_ref, g2_ref, be2_ref, count)
        sc2_ref[...] = sc
        sh2_ref[...] = sh

    @pl.when(ph == 2)
    def _epilogue():
        sc, sh = sc2_ref[...], sh2_ref[...]
        for py in range(2):
            act = _gelu_exact(y2s_ref[i, py].astype(jnp.float32) * sc + sh)
            v = act + yss_ref[i, py].astype(jnp.float32)
            for r in range(hh):
                fr = 2 * r + py
                out_ref[0, :, fr * width:(fr + 1) * width] = (
                    v[:, r * stride:r * stride + width])


def kernel(x, skip, w1, b1, g1, be1, w2, b2, g2, be2, wsx, wss, bs):
    n, c2, hh, ww = x.shape
    _, d, hgt, wid = skip.shape
    p4, p = hh * ww, hgt * wid
    stride = wid + 2
    pp = hh * stride
    m = max(128, pl.cdiv(stride + 1, 128) * 128)
    slen = 2 * m + pp
    bf16, f32 = jnp.bfloat16, jnp.float32

    xf = x.reshape(n, c2, p4)
    sf = skip.reshape(n, d, p)

    wu = w1[:, :, :c2]
    wsk = w1[:, :, c2:]

    def taps(w, dy):
        return jnp.stack([w[(dy + 1) * 3 + (dx + 1)] for dx in (-1, 0, 1)])

    wc1 = jnp.concatenate([
        jnp.concatenate([taps(wsk, 1), taps(wu, 0) + taps(wu, 1),
                         taps(wsk, 0)], axis=2),
        jnp.concatenate([taps(wsk, 0), taps(wu, -1) + taps(wu, 0),
                         taps(wsk, -1)], axis=2)], axis=1).astype(bf16)
    wm1 = jnp.concatenate([taps(wsk, -1), taps(wu, -1)], axis=2).astype(bf16)
    wp1 = jnp.concatenate([taps(wu, 1), taps(wsk, 1)], axis=2).astype(bf16)
    wc2 = jnp.concatenate([
        jnp.concatenate([taps(w2, 1), taps(w2, 0)], axis=2),
        jnp.concatenate([taps(w2, 0), taps(w2, -1)], axis=2)],
        axis=1).astype(bf16)
    wm2 = taps(w2, -1).astype(bf16)
    wp2 = taps(w2, 1).astype(bf16)
    wys0 = jnp.concatenate([wsx, wss], axis=1).astype(bf16)
    wys1 = jnp.concatenate([wss, wsx], axis=1).astype(bf16)

    rr = np.arange(pp) // stride
    cc = np.arange(pp) % stride
    interior = cc < wid
    src = np.where(interior, rr * ww + np.minimum(cc, wid - 1) // 2, -1)
    muph = jnp.asarray(np.arange(p4)[:, None] == src[None, :], bf16)
    maskv = jnp.asarray(interior[:, None], f32)
    gmask = jnp.asarray(interior[None, :], bf16)

    g1c, be1c = g1.reshape(d, 1), be1.reshape(d, 1)
    g2c, be2c = g2.reshape(d, 1), be2.reshape(d, 1)
    c3s = d + c2 + d

    def img0(pidx, ii):
        return (jnp.where(pidx == 0, ii, 0), 0, 0)

    cnst = lambda pidx, ii: (0, 0)
    cnst3 = lambda pidx, ii: (0, 0, 0)

    out = pl.pallas_call(
        partial(_fused, stride=stride, margin=m, pp=pp, hh=hh, width=wid,
                count=float(n * p)),
        grid=(3, n),
        in_specs=[
            pl.BlockSpec((1, c2, p4), img0),
            pl.BlockSpec((1, d, p), img0),
            pl.BlockSpec((p4, pp), cnst),
            pl.BlockSpec((3, 2 * d, c3s), cnst3),
            pl.BlockSpec((3, d, d + c2), cnst3),
            pl.BlockSpec((3, d, c2 + d), cnst3),
            pl.BlockSpec((d, 1), cnst),
            pl.BlockSpec((d, c2 + d), cnst),
            pl.BlockSpec((d, d + c2), cnst),
            pl.BlockSpec((d, 1), cnst),
            pl.BlockSpec((d, 1), cnst),
            pl.BlockSpec((d, 1), cnst),
            pl.BlockSpec((3, 2 * d, 2 * d), cnst3),
            pl.BlockSpec((3, d, d), cnst3),
            pl.BlockSpec((3, d, d), cnst3),
            pl.BlockSpec((d, 1), cnst),
            pl.BlockSpec((d, 1), cnst),
            pl.BlockSpec((d, 1), cnst),
            pl.BlockSpec((pp, 1), cnst),
            pl.BlockSpec((1, pp), cnst),
        ],
        out_specs=pl.BlockSpec(
            (1, d, p), lambda pidx, ii: (jnp.where(pidx == 2, ii, 0), 0, 0)),
        out_shape=jax.ShapeDtypeStruct((n, d, p), f32),
        scratch_shapes=[
            pltpu.VMEM((c3s, slen), bf16),
            pltpu.VMEM((2 * d, slen), bf16),
            pltpu.VMEM((n, 2, d, pp), bf16),
            pltpu.VMEM((n, 2, d, pp), bf16),
            pltpu.VMEM((n, 2, d, pp), bf16),
            pltpu.VMEM((d, 1), f32),
            pltpu.VMEM((d, 1), f32),
            pltpu.VMEM((d, 1), f32),
            pltpu.VMEM((d, 1), f32),
            pltpu.VMEM((d, 1), f32),
            pltpu.VMEM((d, 1), f32),
            pltpu.VMEM((d, 1), f32),
            pltpu.VMEM((d, 1), f32),
        ],
        compiler_params=pltpu.CompilerParams(
            dimension_semantics=("arbitrary", "arbitrary"),
            vmem_limit_bytes=56 * 1024 * 1024),
    )(xf, sf, muph, wc1, wm1, wp1, b1, wys0, wys1, bs, g1c, be1c,
      wc2, wm2, wp2, b2, g2c, be2c, maskv, gmask)

    return out.reshape(n, d, hgt, wid)


# speedup vs baseline: 1.1887x; 1.0447x over previous
from functools import partial

import numpy as np
import jax
import jax.numpy as jnp
from jax import lax
from jax.experimental import pallas as pl
from jax.experimental.pallas import tpu as pltpu

_EPS = 1e-5
_INV_SQRT2 = 0.7071067811865475


def _gelu_exact(v):
    return 0.5 * v * (1.0 + lax.erf(v * _INV_SQRT2))


def _combine_dx(u, b, pp):
    return (u[1] + pltpu.roll(u[2], pp - 1, axis=1)
            + pltpu.roll(u[0], 1, axis=1) + b)


def _zero_margins(slab, rows, m, pp):
    z = jnp.zeros((rows, m), jnp.bfloat16)
    slab[:, 0:m] = z
    slab[:, m + pp:m + pp + m] = z


def _fill_rows(slab, r0, val, nrows, width, stride, m, row0, rstep):
    zg = jnp.zeros((val.shape[0], stride - width), jnp.bfloat16)
    for i in range(nrows):
        r = row0 + i * rstep
        slab[r0:r0 + val.shape[0],
             m + i * stride:m + i * stride + width] = (
            val[:, r * width:(r + 1) * width])
        slab[r0:r0 + val.shape[0],
             m + i * stride + width:m + (i + 1) * stride] = zg


def _bn_from_sums(s_ref, q_ref, g_ref, be_ref, count):
    tot = s_ref[...]
    totsq = q_ref[...]
    mu = tot / count
    var = totsq / count - mu * mu
    inv = lax.rsqrt(jnp.maximum(var, 0.0) + _EPS)
    sc = g_ref[...] * inv
    sh = be_ref[...] - mu * sc
    return sc, sh


def _conv9(mega, wc_ref, wm_ref, wp_ref, lo_rows, hi_rows, b, *,
           stride, m, pp, d):
    c_sl = mega[:, m:m + pp]
    m_sl = mega[0:lo_rows, m - stride:m - stride + pp]
    p_sl = mega[mega.shape[0] - hi_rows:, m + stride:m + stride + pp]
    u0 = [None, None, None]
    u1 = [None, None, None]
    for j in range(3):
        t = jnp.dot(wc_ref[j], c_sl, preferred_element_type=jnp.float32)
        u0[j] = t[:d] + jnp.dot(wm_ref[j], m_sl,
                                preferred_element_type=jnp.float32)
        u1[j] = t[d:] + jnp.dot(wp_ref[j], p_sl,
                                preferred_element_type=jnp.float32)
    return _combine_dx(u0, b, pp), _combine_dx(u1, b, pp)


def _fused(x_ref, skip_ref, muph_ref, wc1_ref, wm1_ref, wp1_ref, b1_ref,
           wys0_ref, wys1_ref, bs_ref, g1_ref, be1_ref,
           wc2_ref, wm2_ref, wp2_ref, b2_ref, g2_ref, be2_ref,
           mv_ref, gm_ref,
           out_ref,
           mega_ref, act_ref,
           y1s_ref, yss_ref, y2s_ref,
           s1a_ref, q1a_ref, s2a_ref, q2a_ref,
           sc1_ref, sh1_ref, sc2_ref, sh2_ref,
           *, stride, margin, pp, hh, width, count):
    ph = pl.program_id(0)
    i = pl.program_id(1)
    nimg = x_ref.shape[0]
    c2, p4 = x_ref.shape[1], x_ref.shape[2]
    d = skip_ref.shape[1]
    m = margin
    mv = mv_ref[...]

    @pl.when(ph == 0)
    def _conv1():
        s = q = None
        for k in range(nimg):
            img = i * nimg + k
            _zero_margins(mega_ref, d + c2 + d, m, pp)

            uph = jnp.dot(x_ref[k].astype(jnp.bfloat16),
                          muph_ref[...], preferred_element_type=jnp.float32)
            mega_ref[d:d + c2, m:m + pp] = uph.astype(jnp.bfloat16)

            sk = skip_ref[k].astype(jnp.bfloat16)
            _fill_rows(mega_ref, 0, sk, hh, width, stride, m, 1, 2)
            _fill_rows(mega_ref, d + c2, sk, hh, width, stride, m, 0, 2)

            raw0, raw1 = _conv9(mega_ref, wc1_ref, wm1_ref, wp1_ref,
                                d + c2, c2 + d, b1_ref[...],
                                stride=stride, m=m, pp=pp, d=d)

            bsv = bs_ref[...]
            yss_ref[img, 0] = (jnp.dot(wys0_ref[...], mega_ref[d:, m:m + pp],
                                       preferred_element_type=jnp.float32)
                               + bsv).astype(jnp.bfloat16)
            yss_ref[img, 1] = (jnp.dot(wys1_ref[...],
                                       mega_ref[:d + c2, m:m + pp],
                                       preferred_element_type=jnp.float32)
                               + bsv).astype(jnp.bfloat16)

            y1s_ref[img, 0] = raw0.astype(jnp.bfloat16)
            y1s_ref[img, 1] = raw1.astype(jnp.bfloat16)
            sk_ = (jnp.dot(raw0, mv, preferred_element_type=jnp.float32)
                   + jnp.dot(raw1, mv, preferred_element_type=jnp.float32))
            qk = (jnp.dot(raw0 * raw0, mv, preferred_element_type=jnp.float32)
                  + jnp.dot(raw1 * raw1, mv,
                            preferred_element_type=jnp.float32))
            s = sk_ if s is None else s + sk_
            q = qk if q is None else q + qk
        s1a_ref[...] = jnp.where(i == 0, s, s1a_ref[...] + s)
        q1a_ref[...] = jnp.where(i == 0, q, q1a_ref[...] + q)

    @pl.when((ph == 1) & (i == 0))
    def _bn1():
        sc, sh = _bn_from_sums(s1a_ref, q1a_ref, g1_ref, be1_ref, count)
        sc1_ref[...] = sc
        sh1_ref[...] = sh

    @pl.when(ph == 1)
    def _conv2():
        s = q = None
        sc, sh, gm = sc1_ref[...], sh1_ref[...], gm_ref[...]
        for k in range(nimg):
            img = i * nimg + k
            _zero_margins(act_ref, 2 * d, m, pp)

            act1 = _gelu_exact(y1s_ref[img, 1].astype(jnp.float32) * sc + sh)
            act_ref[0:d, m:m + pp] = act1.astype(jnp.bfloat16) * gm
            act0 = _gelu_exact(y1s_ref[img, 0].astype(jnp.float32) * sc + sh)
            act_ref[d:, m:m + pp] = act0.astype(jnp.bfloat16) * gm

            raw0, raw1 = _conv9(act_ref, wc2_ref, wm2_ref, wp2_ref,
                                d, d, b2_ref[...],
                                stride=stride, m=m, pp=pp, d=d)

            y2s_ref[img, 0] = raw0.astype(jnp.bfloat16)
            y2s_ref[img, 1] = raw1.astype(jnp.bfloat16)
            sk_ = (jnp.dot(raw0, mv, preferred_element_type=jnp.float32)
                   + jnp.dot(raw1, mv, preferred_element_type=jnp.float32))
            qk = (jnp.dot(raw0 * raw0, mv, preferred_element_type=jnp.float32)
                  + jnp.dot(raw1 * raw1, mv,
                            preferred_element_type=jnp.float32))
            s = sk_ if s is None else s + sk_
            q = qk if q is None else q + qk
        s2a_ref[...] = jnp.where(i == 0, s, s2a_ref[...] + s)
        q2a_ref[...] = jnp.where(i == 0, q, q2a_ref[...] + q)

    @pl.when((ph == 2) & (i == 0))
    def _bn2():
        sc, sh = _bn_from_sums(s2a_ref, q2a_ref, g2_ref, be2_ref, count)
        sc2_ref[...] = sc
        sh2_ref[...] = sh

    @pl.when(ph == 2)
    def _epilogue():
        sc, sh = sc2_ref[...], sh2_ref[...]
        for k in range(nimg):
            img = i * nimg + k
            for py in range(2):
                act = _gelu_exact(y2s_ref[img, py].astype(jnp.float32)
                                  * sc + sh)
                v = act + yss_ref[img, py].astype(jnp.float32)
                for r in range(hh):
                    fr = 2 * r + py
                    out_ref[k, :, fr * width:(fr + 1) * width] = (
                        v[:, r * stride:r * stride + width])


def kernel(x, skip, w1, b1, g1, be1, w2, b2, g2, be2, wsx, wss, bs):
    n, c2, hh, ww = x.shape
    _, d, hgt, wid = skip.shape
    p4, p = hh * ww, hgt * wid
    stride = wid + 2
    pp = hh * stride
    m = max(128, pl.cdiv(stride + 1, 128) * 128)
    slen = 2 * m + pp
    bf16, f32 = jnp.bfloat16, jnp.float32

    xf = x.reshape(n, c2, p4)
    sf = skip.reshape(n, d, p)

    wu = w1[:, :, :c2]
    wsk = w1[:, :, c2:]

    def taps(w, dy):
        return jnp.stack([w[(dy + 1) * 3 + (dx + 1)] for dx in (-1, 0, 1)])

    wc1 = jnp.concatenate([
        jnp.concatenate([taps(wsk, 1), taps(wu, 0) + taps(wu, 1),
                         taps(wsk, 0)], axis=2),
        jnp.concatenate([taps(wsk, 0), taps(wu, -1) + taps(wu, 0),
                         taps(wsk, -1)], axis=2)], axis=1).astype(bf16)
    wm1 = jnp.concatenate([taps(wsk, -1), taps(wu, -1)], axis=2).astype(bf16)
    wp1 = jnp.concatenate([taps(wu, 1), taps(wsk, 1)], axis=2).astype(bf16)
    wc2 = jnp.concatenate([
        jnp.concatenate([taps(w2, 1), taps(w2, 0)], axis=2),
        jnp.concatenate([taps(w2, 0), taps(w2, -1)], axis=2)],
        axis=1).astype(bf16)
    wm2 = taps(w2, -1).astype(bf16)
    wp2 = taps(w2, 1).astype(bf16)
    wys0 = jnp.concatenate([wsx, wss], axis=1).astype(bf16)
    wys1 = jnp.concatenate([wss, wsx], axis=1).astype(bf16)

    rr = np.arange(pp) // stride
    cc = np.arange(pp) % stride
    interior = cc < wid
    src = np.where(interior, rr * ww + np.minimum(cc, wid - 1) // 2, -1)
    muph = jnp.asarray(np.arange(p4)[:, None] == src[None, :], bf16)
    maskv = jnp.asarray(interior[:, None], f32)
    gmask = jnp.asarray(interior[None, :], bf16)

    g1c, be1c = g1.reshape(d, 1), be1.reshape(d, 1)
    g2c, be2c = g2.reshape(d, 1), be2.reshape(d, 1)
    c3s = d + c2 + d

    imgs = 2 if n % 2 == 0 else 1

    def img0(pidx, ii):
        return (jnp.where(pidx == 0, ii, 0), 0, 0)

    cnst = lambda pidx, ii: (0, 0)
    cnst3 = lambda pidx, ii: (0, 0, 0)

    out = pl.pallas_call(
        partial(_fused, stride=stride, margin=m, pp=pp, hh=hh, width=wid,
                count=float(n * p)),
        grid=(3, n // imgs),
        in_specs=[
            pl.BlockSpec((imgs, c2, p4), img0),
            pl.BlockSpec((imgs, d, p), img0),
            pl.BlockSpec((p4, pp), cnst),
            pl.BlockSpec((3, 2 * d, c3s), cnst3),
            pl.BlockSpec((3, d, d + c2), cnst3),
            pl.BlockSpec((3, d, c2 + d), cnst3),
            pl.BlockSpec((d, 1), cnst),
            pl.BlockSpec((d, c2 + d), cnst),
            pl.BlockSpec((d, d + c2), cnst),
            pl.BlockSpec((d, 1), cnst),
            pl.BlockSpec((d, 1), cnst),
            pl.BlockSpec((d, 1), cnst),
            pl.BlockSpec((3, 2 * d, 2 * d), cnst3),
            pl.BlockSpec((3, d, d), cnst3),
            pl.BlockSpec((3, d, d), cnst3),
            pl.BlockSpec((d, 1), cnst),
            pl.BlockSpec((d, 1), cnst),
            pl.BlockSpec((d, 1), cnst),
            pl.BlockSpec((pp, 1), cnst),
            pl.BlockSpec((1, pp), cnst),
        ],
        out_specs=pl.BlockSpec(
            (imgs, d, p),
            lambda pidx, ii: (jnp.where(pidx == 2, ii, 0), 0, 0)),
        out_shape=jax.ShapeDtypeStruct((n, d, p), f32),
        scratch_shapes=[
            pltpu.VMEM((c3s, slen), bf16),
            pltpu.VMEM((2 * d, slen), bf16),
            pltpu.VMEM((n, 2, d, pp), bf16),
            pltpu.VMEM((n, 2, d, pp), bf16),
            pltpu.VMEM((n, 2, d, pp), bf16),
            pltpu.VMEM((d, 1), f32),
            pltpu.VMEM((d, 1), f32),
            pltpu.VMEM((d, 1), f32),
            pltpu.VMEM((d, 1), f32),
            pltpu.VMEM((d, 1), f32),
            pltpu.VMEM((d, 1), f32),
            pltpu.VMEM((d, 1), f32),
            pltpu.VMEM((d, 1), f32),
        ],
        compiler_params=pltpu.CompilerParams(
            dimension_semantics=("arbitrary", "arbitrary"),
            vmem_limit_bytes=56 * 1024 * 1024),
    )(xf, sf, muph, wc1, wm1, wp1, b1, wys0, wys1, bs, g1c, be1c,
      wc2, wm2, wp2, b2, g2c, be2c, maskv, gmask)

    return out.reshape(n, d, hgt, wid)


# speedup vs baseline: 1.2143x; 1.0215x over previous
from functools import partial

import numpy as np
import jax
import jax.numpy as jnp
from jax import lax
from jax.experimental import pallas as pl
from jax.experimental.pallas import tpu as pltpu

_EPS = 1e-5
_INV_SQRT2 = 0.7071067811865475


def _gelu_exact(v):
    return 0.5 * v * (1.0 + lax.erf(v * _INV_SQRT2))


def _combine_dx(u, b, pp):
    c = u[1].astype(jnp.bfloat16)
    l = u[2].astype(jnp.bfloat16)
    r = u[0].astype(jnp.bfloat16)
    left = jnp.concatenate([l[:, 1:], l[:, :1]], axis=1)
    right = jnp.concatenate([r[:, -1:], r[:, :-1]], axis=1)
    return c + left + right + b


def _zero_margins(slab, rows, m, pp):
    z = jnp.zeros((rows, m), jnp.bfloat16)
    slab[:, 0:m] = z
    slab[:, m + pp:m + pp + m] = z


def _fill_rows(slab, r0, val, nrows, width, stride, m, row0, rstep):
    zg = jnp.zeros((val.shape[0], stride - width), jnp.bfloat16)
    for i in range(nrows):
        r = row0 + i * rstep
        slab[r0:r0 + val.shape[0],
             m + i * stride:m + i * stride + width] = (
            val[:, r * width:(r + 1) * width])
        slab[r0:r0 + val.shape[0],
             m + i * stride + width:m + (i + 1) * stride] = zg


def _bn_from_sums(s_ref, q_ref, g_ref, be_ref, count):
    tot = s_ref[...]
    totsq = q_ref[...]
    mu = tot / count
    var = totsq / count - mu * mu
    inv = lax.rsqrt(jnp.maximum(var, 0.0) + _EPS)
    sc = g_ref[...] * inv
    sh = be_ref[...] - mu * sc
    return sc, sh


def _conv9(mega, wc_ref, wm_ref, wp_ref, lo_rows, hi_rows, b, *,
           stride, m, pp, d):
    c_sl = mega[:, m:m + pp]
    m_sl = mega[0:lo_rows, m - stride:m - stride + pp]
    p_sl = mega[mega.shape[0] - hi_rows:, m + stride:m + stride + pp]
    u0 = [None, None, None]
    u1 = [None, None, None]
    for j in range(3):
        t = jnp.dot(wc_ref[j], c_sl, preferred_element_type=jnp.float32)
        u0[j] = t[:d] + jnp.dot(wm_ref[j], m_sl,
                                preferred_element_type=jnp.float32)
        u1[j] = t[d:] + jnp.dot(wp_ref[j], p_sl,
                                preferred_element_type=jnp.float32)
    return _combine_dx(u0, b, pp), _combine_dx(u1, b, pp)


def _fused(x_ref, skip_ref, muph_ref, wc1_ref, wm1_ref, wp1_ref, b1_ref,
           wys0_ref, wys1_ref, bs_ref, g1_ref, be1_ref,
           wc2_ref, wm2_ref, wp2_ref, b2_ref, g2_ref, be2_ref,
           mv_ref, gm_ref,
           out_ref,
           mega_ref, act_ref,
           y1s_ref, yss_ref, y2s_ref,
           s1a_ref, q1a_ref, s2a_ref, q2a_ref,
           sc1_ref, sh1_ref, sc2_ref, sh2_ref,
           *, stride, margin, pp, hh, width, count):
    ph = pl.program_id(0)
    i = pl.program_id(1)
    nimg = x_ref.shape[0]
    c2, p4 = x_ref.shape[1], x_ref.shape[2]
    d = skip_ref.shape[1]
    m = margin
    mv = mv_ref[...]

    @pl.when(ph == 0)
    def _conv1():
        s = q = None
        for k in range(nimg):
            img = i * nimg + k
            _zero_margins(mega_ref, d + c2 + d, m, pp)

            uph = jnp.dot(x_ref[k].astype(jnp.bfloat16),
                          muph_ref[...], preferred_element_type=jnp.float32)
            mega_ref[d:d + c2, m:m + pp] = uph.astype(jnp.bfloat16)

            sk = skip_ref[k].astype(jnp.bfloat16)
            _fill_rows(mega_ref, 0, sk, hh, width, stride, m, 1, 2)
            _fill_rows(mega_ref, d + c2, sk, hh, width, stride, m, 0, 2)

            raw0, raw1 = _conv9(mega_ref, wc1_ref, wm1_ref, wp1_ref,
                                d + c2, c2 + d, b1_ref[...],
                                stride=stride, m=m, pp=pp, d=d)

            bsv = bs_ref[...]
            yss_ref[img, 0] = (jnp.dot(wys0_ref[...], mega_ref[d:, m:m + pp],
                                       preferred_element_type=jnp.float32)
                               + bsv).astype(jnp.bfloat16)
            yss_ref[img, 1] = (jnp.dot(wys1_ref[...],
                                       mega_ref[:d + c2, m:m + pp],
                                       preferred_element_type=jnp.float32)
                               + bsv).astype(jnp.bfloat16)

            y1s_ref[img, 0] = raw0
            y1s_ref[img, 1] = raw1
            sk_ = (jnp.dot(raw0, mv, preferred_element_type=jnp.float32)
                   + jnp.dot(raw1, mv, preferred_element_type=jnp.float32))
            qk = (jnp.dot(raw0 * raw0, mv, preferred_element_type=jnp.float32)
                  + jnp.dot(raw1 * raw1, mv,
                            preferred_element_type=jnp.float32))
            s = sk_ if s is None else s + sk_
            q = qk if q is None else q + qk
        s1a_ref[...] = jnp.where(i == 0, s, s1a_ref[...] + s)
        q1a_ref[...] = jnp.where(i == 0, q, q1a_ref[...] + q)

    @pl.when((ph == 1) & (i == 0))
    def _bn1():
        sc, sh = _bn_from_sums(s1a_ref, q1a_ref, g1_ref, be1_ref, count)
        sc1_ref[...] = sc
        sh1_ref[...] = sh

    @pl.when(ph == 1)
    def _conv2():
        s = q = None
        sc, sh, gm = sc1_ref[...], sh1_ref[...], gm_ref[...]
        for k in range(nimg):
            img = i * nimg + k
            _zero_margins(act_ref, 2 * d, m, pp)

            act1 = _gelu_exact(y1s_ref[img, 1].astype(jnp.float32) * sc + sh)
            act_ref[0:d, m:m + pp] = act1.astype(jnp.bfloat16) * gm
            act0 = _gelu_exact(y1s_ref[img, 0].astype(jnp.float32) * sc + sh)
            act_ref[d:, m:m + pp] = act0.astype(jnp.bfloat16) * gm

            raw0, raw1 = _conv9(act_ref, wc2_ref, wm2_ref, wp2_ref,
                                d, d, b2_ref[...],
                                stride=stride, m=m, pp=pp, d=d)

            y2s_ref[img, 0] = raw0
            y2s_ref[img, 1] = raw1
            sk_ = (jnp.dot(raw0, mv, preferred_element_type=jnp.float32)
                   + jnp.dot(raw1, mv, preferred_element_type=jnp.float32))
            qk = (jnp.dot(raw0 * raw0, mv, preferred_element_type=jnp.float32)
                  + jnp.dot(raw1 * raw1, mv,
                            preferred_element_type=jnp.float32))
            s = sk_ if s is None else s + sk_
            q = qk if q is None else q + qk
        s2a_ref[...] = jnp.where(i == 0, s, s2a_ref[...] + s)
        q2a_ref[...] = jnp.where(i == 0, q, q2a_ref[...] + q)

    @pl.when((ph == 2) & (i == 0))
    def _bn2():
        sc, sh = _bn_from_sums(s2a_ref, q2a_ref, g2_ref, be2_ref, count)
        sc2_ref[...] = sc
        sh2_ref[...] = sh

    @pl.when(ph == 2)
    def _epilogue():
        sc, sh = sc2_ref[...], sh2_ref[...]
        for k in range(nimg):
            img = i * nimg + k
            for py in range(2):
                act = _gelu_exact(y2s_ref[img, py].astype(jnp.float32)
                                  * sc + sh)
                v = act + yss_ref[img, py].astype(jnp.float32)
                for r in range(hh):
                    fr = 2 * r + py
                    out_ref[k, :, fr * width:(fr + 1) * width] = (
                        v[:, r * stride:r * stride + width])


def kernel(x, skip, w1, b1, g1, be1, w2, b2, g2, be2, wsx, wss, bs):
    n, c2, hh, ww = x.shape
    _, d, hgt, wid = skip.shape
    p4, p = hh * ww, hgt * wid
    stride = wid + 2
    pp = hh * stride
    m = max(128, pl.cdiv(stride + 1, 128) * 128)
    slen = 2 * m + pp
    bf16, f32 = jnp.bfloat16, jnp.float32

    xf = x.reshape(n, c2, p4)
    sf = skip.reshape(n, d, p)

    wu = w1[:, :, :c2]
    wsk = w1[:, :, c2:]

    def taps(w, dy):
        return jnp.stack([w[(dy + 1) * 3 + (dx + 1)] for dx in (-1, 0, 1)])

    wc1 = jnp.concatenate([
        jnp.concatenate([taps(wsk, 1), taps(wu, 0) + taps(wu, 1),
                         taps(wsk, 0)], axis=2),
        jnp.concatenate([taps(wsk, 0), taps(wu, -1) + taps(wu, 0),
                         taps(wsk, -1)], axis=2)], axis=1).astype(bf16)
    wm1 = jnp.concatenate([taps(wsk, -1), taps(wu, -1)], axis=2).astype(bf16)
    wp1 = jnp.concatenate([taps(wu, 1), taps(wsk, 1)], axis=2).astype(bf16)
    wc2 = jnp.concatenate([
        jnp.concatenate([taps(w2, 1), taps(w2, 0)], axis=2),
        jnp.concatenate([taps(w2, 0), taps(w2, -1)], axis=2)],
        axis=1).astype(bf16)
    wm2 = taps(w2, -1).astype(bf16)
    wp2 = taps(w2, 1).astype(bf16)
    wys0 = jnp.concatenate([wsx, wss], axis=1).astype(bf16)
    wys1 = jnp.concatenate([wss, wsx], axis=1).astype(bf16)

    rr = np.arange(pp) // stride
    cc = np.arange(pp) % stride
    interior = cc < wid
    src = np.where(interior, rr * ww + np.minimum(cc, wid - 1) // 2, -1)
    muph = jnp.asarray(np.arange(p4)[:, None] == src[None, :], bf16)
    maskv = jnp.asarray(interior[:, None], bf16)
    gmask = jnp.asarray(interior[None, :], bf16)

    g1c, be1c = g1.reshape(d, 1), be1.reshape(d, 1)
    g2c, be2c = g2.reshape(d, 1), be2.reshape(d, 1)
    b1b, b2b = b1.astype(bf16), b2.astype(bf16)
    c3s = d + c2 + d

    imgs = 2 if n % 2 == 0 else 1

    def img0(pidx, ii):
        return (jnp.where(pidx == 0, ii, 0), 0, 0)

    cnst = lambda pidx, ii: (0, 0)
    cnst3 = lambda pidx, ii: (0, 0, 0)

    out = pl.pallas_call(
        partial(_fused, stride=stride, margin=m, pp=pp, hh=hh, width=wid,
                count=float(n * p)),
        grid=(3, n // imgs),
        in_specs=[
            pl.BlockSpec((imgs, c2, p4), img0),
            pl.BlockSpec((imgs, d, p), img0),
            pl.BlockSpec((p4, pp), cnst),
            pl.BlockSpec((3, 2 * d, c3s), cnst3),
            pl.BlockSpec((3, d, d + c2), cnst3),
            pl.BlockSpec((3, d, c2 + d), cnst3),
            pl.BlockSpec((d, 1), cnst),
            pl.BlockSpec((d, c2 + d), cnst),
            pl.BlockSpec((d, d + c2), cnst),
            pl.BlockSpec((d, 1), cnst),
            pl.BlockSpec((d, 1), cnst),
            pl.BlockSpec((d, 1), cnst),
            pl.BlockSpec((3, 2 * d, 2 * d), cnst3),
            pl.BlockSpec((3, d, d), cnst3),
            pl.BlockSpec((3, d, d), cnst3),
            pl.BlockSpec((d, 1), cnst),
            pl.BlockSpec((d, 1), cnst),
            pl.BlockSpec((d, 1), cnst),
            pl.BlockSpec((pp, 1), cnst),
            pl.BlockSpec((1, pp), cnst),
        ],
        out_specs=pl.BlockSpec(
            (imgs, d, p),
            lambda pidx, ii: (jnp.where(pidx == 2, ii, 0), 0, 0)),
        out_shape=jax.ShapeDtypeStruct((n, d, p), f32),
        scratch_shapes=[
            pltpu.VMEM((c3s, slen), bf16),
            pltpu.VMEM((2 * d, slen), bf16),
            pltpu.VMEM((n, 2, d, pp), bf16),
            pltpu.VMEM((n, 2, d, pp), bf16),
            pltpu.VMEM((n, 2, d, pp), bf16),
            pltpu.VMEM((d, 1), f32),
            pltpu.VMEM((d, 1), f32),
            pltpu.VMEM((d, 1), f32),
            pltpu.VMEM((d, 1), f32),
            pltpu.VMEM((d, 1), f32),
            pltpu.VMEM((d, 1), f32),
            pltpu.VMEM((d, 1), f32),
            pltpu.VMEM((d, 1), f32),
        ],
        compiler_params=pltpu.CompilerParams(
            dimension_semantics=("arbitrary", "arbitrary"),
            vmem_limit_bytes=56 * 1024 * 1024),
    )(xf, sf, muph, wc1, wm1, wp1, b1b, wys0, wys1, bs, g1c, be1c,
      wc2, wm2, wp2, b2b, g2c, be2c, maskv, gmask)

    return out.reshape(n, d, hgt, wid)


# speedup vs baseline: 1.2610x; 1.0385x over previous
from functools import partial

import numpy as np
import jax
import jax.numpy as jnp
from jax import lax
from jax.experimental import pallas as pl
from jax.experimental.pallas import tpu as pltpu

_EPS = 1e-5
_INV_SQRT2 = 0.7071067811865475


def _gelu_exact(v):
    return 0.5 * v * (1.0 + lax.erf(v * _INV_SQRT2))


def _combine_dx(u, b, pp):
    c = u[1].astype(jnp.bfloat16)
    l = u[2].astype(jnp.bfloat16)
    r = u[0].astype(jnp.bfloat16)
    left = jnp.concatenate([l[:, 1:], l[:, :1]], axis=1)
    right = jnp.concatenate([r[:, -1:], r[:, :-1]], axis=1)
    return c + left + right + b


def _zero_margins(slab, rows, m, pp):
    z = jnp.zeros((rows, m), jnp.bfloat16)
    slab[:, 0:m] = z
    slab[:, m + pp:m + pp + m] = z


def _fill_rows(slab, r0, val, nrows, width, stride, m, row0, rstep):
    zg = jnp.zeros((val.shape[0], stride - width), jnp.bfloat16)
    for i in range(nrows):
        r = row0 + i * rstep
        slab[r0:r0 + val.shape[0],
             m + i * stride:m + i * stride + width] = (
            val[:, r * width:(r + 1) * width])
        slab[r0:r0 + val.shape[0],
             m + i * stride + width:m + (i + 1) * stride] = zg


def _bn_from_sums(s_ref, q_ref, g_ref, be_ref, count):
    tot = s_ref[...]
    totsq = q_ref[...]
    mu = tot / count
    var = totsq / count - mu * mu
    inv = lax.rsqrt(jnp.maximum(var, 0.0) + _EPS)
    sc = g_ref[...] * inv
    sh = be_ref[...] - mu * sc
    return sc, sh


def _conv9(mega, wc_ref, wm_ref, wp_ref, lo_rows, hi_rows, b, *,
           stride, m, pp, d):
    c_sl = mega[:, m:m + pp]
    m_sl = mega[0:lo_rows, m - stride:m - stride + pp]
    p_sl = mega[mega.shape[0] - hi_rows:, m + stride:m + stride + pp]
    u0 = [None, None, None]
    u1 = [None, None, None]
    for j in range(3):
        t = jnp.dot(wc_ref[j], c_sl, preferred_element_type=jnp.float32)
        u0[j] = t[:d] + jnp.dot(wm_ref[j], m_sl,
                                preferred_element_type=jnp.float32)
        u1[j] = t[d:] + jnp.dot(wp_ref[j], p_sl,
                                preferred_element_type=jnp.float32)
    return _combine_dx(u0, b, pp), _combine_dx(u1, b, pp)


def _fused(x_ref, skip_ref, muph_ref, wc1_ref, wm1_ref, wp1_ref, b1_ref,
           wys0_ref, wys1_ref, bs_ref, g1_ref, be1_ref,
           wc2_ref, wm2_ref, wp2_ref, b2_ref, g2_ref, be2_ref,
           mv_ref, gm_ref,
           out_ref,
           mega_ref, act_ref,
           y1s_ref, yss_ref,
           s1a_ref, q1a_ref, s2a_ref, q2a_ref,
           sc1_ref, sh1_ref, sc2_ref, sh2_ref,
           *, stride, margin, pp, hh, width, count):
    ph = pl.program_id(0)
    i = pl.program_id(1)
    nimg = x_ref.shape[0]
    c2, p4 = x_ref.shape[1], x_ref.shape[2]
    d = skip_ref.shape[1]
    m = margin
    mv = mv_ref[...]

    @pl.when(ph == 0)
    def _conv1():
        s = q = None
        for k in range(nimg):
            img = i * nimg + k
            _zero_margins(mega_ref, d + c2 + d, m, pp)

            uph = jnp.dot(x_ref[k].astype(jnp.bfloat16),
                          muph_ref[...], preferred_element_type=jnp.float32)
            mega_ref[d:d + c2, m:m + pp] = uph.astype(jnp.bfloat16)

            sk = skip_ref[k].astype(jnp.bfloat16)
            _fill_rows(mega_ref, 0, sk, hh, width, stride, m, 1, 2)
            _fill_rows(mega_ref, d + c2, sk, hh, width, stride, m, 0, 2)

            raw0, raw1 = _conv9(mega_ref, wc1_ref, wm1_ref, wp1_ref,
                                d + c2, c2 + d, b1_ref[...],
                                stride=stride, m=m, pp=pp, d=d)

            bsv = bs_ref[...]
            yss_ref[img, 0] = (jnp.dot(wys0_ref[...], mega_ref[d:, m:m + pp],
                                       preferred_element_type=jnp.float32)
                               + bsv).astype(jnp.bfloat16)
            yss_ref[img, 1] = (jnp.dot(wys1_ref[...],
                                       mega_ref[:d + c2, m:m + pp],
                                       preferred_element_type=jnp.float32)
                               + bsv).astype(jnp.bfloat16)

            y1s_ref[img, 0] = raw0
            y1s_ref[img, 1] = raw1
            sk_ = (jnp.dot(raw0, mv, preferred_element_type=jnp.float32)
                   + jnp.dot(raw1, mv, preferred_element_type=jnp.float32))
            qk = (jnp.dot(raw0 * raw0, mv, preferred_element_type=jnp.float32)
                  + jnp.dot(raw1 * raw1, mv,
                            preferred_element_type=jnp.float32))
            s = sk_ if s is None else s + sk_
            q = qk if q is None else q + qk
        s1a_ref[...] = jnp.where(i == 0, s, s1a_ref[...] + s)
        q1a_ref[...] = jnp.where(i == 0, q, q1a_ref[...] + q)

    @pl.when((ph == 1) & (i == 0))
    def _bn1():
        sc, sh = _bn_from_sums(s1a_ref, q1a_ref, g1_ref, be1_ref, count)
        sc1_ref[...] = sc
        sh1_ref[...] = sh

    @pl.when(ph == 1)
    def _conv2():
        s = q = None
        sc, sh, gm = sc1_ref[...], sh1_ref[...], gm_ref[...]
        for k in range(nimg):
            img = i * nimg + k
            _zero_margins(act_ref, 2 * d, m, pp)

            act1 = _gelu_exact(y1s_ref[img, 1].astype(jnp.float32) * sc + sh)
            act_ref[0:d, m:m + pp] = act1.astype(jnp.bfloat16) * gm
            act0 = _gelu_exact(y1s_ref[img, 0].astype(jnp.float32) * sc + sh)
            act_ref[d:, m:m + pp] = act0.astype(jnp.bfloat16) * gm

            raw0, raw1 = _conv9(act_ref, wc2_ref, wm2_ref, wp2_ref,
                                d, d, b2_ref[...],
                                stride=stride, m=m, pp=pp, d=d)

            y1s_ref[img, 0] = raw0
            y1s_ref[img, 1] = raw1
            sk_ = (jnp.dot(raw0, mv, preferred_element_type=jnp.float32)
                   + jnp.dot(raw1, mv, preferred_element_type=jnp.float32))
            qk = (jnp.dot(raw0 * raw0, mv, preferred_element_type=jnp.float32)
                  + jnp.dot(raw1 * raw1, mv,
                            preferred_element_type=jnp.float32))
            s = sk_ if s is None else s + sk_
            q = qk if q is None else q + qk
        s2a_ref[...] = jnp.where(i == 0, s, s2a_ref[...] + s)
        q2a_ref[...] = jnp.where(i == 0, q, q2a_ref[...] + q)

    @pl.when((ph == 2) & (i == 0))
    def _bn2():
        sc, sh = _bn_from_sums(s2a_ref, q2a_ref, g2_ref, be2_ref, count)
        sc2_ref[...] = sc
        sh2_ref[...] = sh

    @pl.when(ph == 2)
    def _epilogue():
        sc, sh = sc2_ref[...], sh2_ref[...]
        for k in range(nimg):
            img = i * nimg + k
            for py in range(2):
                act = _gelu_exact(y1s_ref[img, py].astype(jnp.float32)
                                  * sc + sh)
                v = act + yss_ref[img, py].astype(jnp.float32)
                for r in range(hh):
                    fr = 2 * r + py
                    out_ref[k, :, fr * width:(fr + 1) * width] = (
                        v[:, r * stride:r * stride + width])


def kernel(x, skip, w1, b1, g1, be1, w2, b2, g2, be2, wsx, wss, bs):
    n, c2, hh, ww = x.shape
    _, d, hgt, wid = skip.shape
    p4, p = hh * ww, hgt * wid
    stride = wid + 2
    pp = hh * stride
    m = max(128, pl.cdiv(stride + 1, 128) * 128)
    slen = 2 * m + pp
    bf16, f32 = jnp.bfloat16, jnp.float32

    xf = x.reshape(n, c2, p4)
    sf = skip.reshape(n, d, p)

    wu = w1[:, :, :c2]
    wsk = w1[:, :, c2:]

    def taps(w, dy):
        return jnp.stack([w[(dy + 1) * 3 + (dx + 1)] for dx in (-1, 0, 1)])

    wc1 = jnp.concatenate([
        jnp.concatenate([taps(wsk, 1), taps(wu, 0) + taps(wu, 1),
                         taps(wsk, 0)], axis=2),
        jnp.concatenate([taps(wsk, 0), taps(wu, -1) + taps(wu, 0),
                         taps(wsk, -1)], axis=2)], axis=1).astype(bf16)
    wm1 = jnp.concatenate([taps(wsk, -1), taps(wu, -1)], axis=2).astype(bf16)
    wp1 = jnp.concatenate([taps(wu, 1), taps(wsk, 1)], axis=2).astype(bf16)
    wc2 = jnp.concatenate([
        jnp.concatenate([taps(w2, 1), taps(w2, 0)], axis=2),
        jnp.concatenate([taps(w2, 0), taps(w2, -1)], axis=2)],
        axis=1).astype(bf16)
    wm2 = taps(w2, -1).astype(bf16)
    wp2 = taps(w2, 1).astype(bf16)
    wys0 = jnp.concatenate([wsx, wss], axis=1).astype(bf16)
    wys1 = jnp.concatenate([wss, wsx], axis=1).astype(bf16)

    rr = np.arange(pp) // stride
    cc = np.arange(pp) % stride
    interior = cc < wid
    src = np.where(interior, rr * ww + np.minimum(cc, wid - 1) // 2, -1)
    muph = jnp.asarray(np.arange(p4)[:, None] == src[None, :], bf16)
    maskv = jnp.asarray(interior[:, None], bf16)
    gmask = jnp.asarray(interior[None, :], bf16)

    g1c, be1c = g1.reshape(d, 1), be1.reshape(d, 1)
    g2c, be2c = g2.reshape(d, 1), be2.reshape(d, 1)
    b1b, b2b = b1.astype(bf16), b2.astype(bf16)
    c3s = d + c2 + d

    imgs = next(k for k in (4, 2, 1) if n % k == 0)

    def img0(pidx, ii):
        return (jnp.where(pidx == 0, ii, 0), 0, 0)

    cnst = lambda pidx, ii: (0, 0)
    cnst3 = lambda pidx, ii: (0, 0, 0)

    out = pl.pallas_call(
        partial(_fused, stride=stride, margin=m, pp=pp, hh=hh, width=wid,
                count=float(n * p)),
        grid=(3, n // imgs),
        in_specs=[
            pl.BlockSpec((imgs, c2, p4), img0),
            pl.BlockSpec((imgs, d, p), img0),
            pl.BlockSpec((p4, pp), cnst),
            pl.BlockSpec((3, 2 * d, c3s), cnst3),
            pl.BlockSpec((3, d, d + c2), cnst3),
            pl.BlockSpec((3, d, c2 + d), cnst3),
            pl.BlockSpec((d, 1), cnst),
            pl.BlockSpec((d, c2 + d), cnst),
            pl.BlockSpec((d, d + c2), cnst),
            pl.BlockSpec((d, 1), cnst),
            pl.BlockSpec((d, 1), cnst),
            pl.BlockSpec((d, 1), cnst),
            pl.BlockSpec((3, 2 * d, 2 * d), cnst3),
            pl.BlockSpec((3, d, d), cnst3),
            pl.BlockSpec((3, d, d), cnst3),
            pl.BlockSpec((d, 1), cnst),
            pl.BlockSpec((d, 1), cnst),
            pl.BlockSpec((d, 1), cnst),
            pl.BlockSpec((pp, 1), cnst),
            pl.BlockSpec((1, pp), cnst),
        ],
        out_specs=pl.BlockSpec(
            (imgs, d, p),
            lambda pidx, ii: (jnp.where(pidx == 2, ii, 0), 0, 0)),
        out_shape=jax.ShapeDtypeStruct((n, d, p), f32),
        scratch_shapes=[
            pltpu.VMEM((c3s, slen), bf16),
            pltpu.VMEM((2 * d, slen), bf16),
            pltpu.VMEM((n, 2, d, pp), bf16),
            pltpu.VMEM((n, 2, d, pp), bf16),
            pltpu.VMEM((d, 1), f32),
            pltpu.VMEM((d, 1), f32),
            pltpu.VMEM((d, 1), f32),
            pltpu.VMEM((d, 1), f32),
            pltpu.VMEM((d, 1), f32),
            pltpu.VMEM((d, 1), f32),
            pltpu.VMEM((d, 1), f32),
            pltpu.VMEM((d, 1), f32),
        ],
        compiler_params=pltpu.CompilerParams(
            dimension_semantics=("arbitrary", "arbitrary"),
            vmem_limit_bytes=56 * 1024 * 1024),
    )(xf, sf, muph, wc1, wm1, wp1, b1b, wys0, wys1, bs, g1c, be1c,
      wc2, wm2, wp2, b2b, g2c, be2c, maskv, gmask)

    return out.reshape(n, d, hgt, wid)


# speedup vs baseline: 1.2639x; 1.0023x over previous
from functools import partial

import numpy as np
import jax
import jax.numpy as jnp
from jax import lax
from jax.experimental import pallas as pl
from jax.experimental.pallas import tpu as pltpu

_EPS = 1e-5
_INV_SQRT2 = 0.7071067811865475


def _gelu_exact(v):
    return 0.5 * v * (1.0 + lax.erf(v * _INV_SQRT2))


def _combine_dx(u, b, pp):
    c = u[1].astype(jnp.bfloat16)
    l = u[2].astype(jnp.bfloat16)
    r = u[0].astype(jnp.bfloat16)
    left = jnp.concatenate([l[:, 1:], l[:, :1]], axis=1)
    right = jnp.concatenate([r[:, -1:], r[:, :-1]], axis=1)
    return c + left + right + b


def _zero_margins(slab, rows, m, pp):
    z = jnp.zeros((rows, m), jnp.bfloat16)
    slab[:, 0:m] = z
    slab[:, m + pp:m + pp + m] = z


def _fill_rows(slab, r0, val, nrows, width, stride, m, row0, rstep):
    zg = jnp.zeros((val.shape[0], stride - width), jnp.bfloat16)
    for i in range(nrows):
        r = row0 + i * rstep
        slab[r0:r0 + val.shape[0],
             m + i * stride:m + i * stride + width] = (
            val[:, r * width:(r + 1) * width])
        slab[r0:r0 + val.shape[0],
             m + i * stride + width:m + (i + 1) * stride] = zg


def _bn_from_sums(s_ref, q_ref, g_ref, be_ref, count):
    tot = s_ref[...]
    totsq = q_ref[...]
    mu = tot / count
    var = totsq / count - mu * mu
    inv = lax.rsqrt(jnp.maximum(var, 0.0) + _EPS)
    sc = g_ref[...] * inv
    sh = be_ref[...] - mu * sc
    return sc, sh


def _conv9(mega, wc_ref, wm_ref, wp_ref, lo_rows, hi_rows, b, *,
           stride, m, pp, d):
    c_sl = mega[:, m:m + pp]
    m_sl = mega[0:lo_rows, m - stride:m - stride + pp]
    p_sl = mega[mega.shape[0] - hi_rows:, m + stride:m + stride + pp]
    u0 = [None, None, None]
    u1 = [None, None, None]
    for j in range(3):
        t = jnp.dot(wc_ref[j], c_sl, preferred_element_type=jnp.float32)
        u0[j] = t[:d] + jnp.dot(wm_ref[j], m_sl,
                                preferred_element_type=jnp.float32)
        u1[j] = t[d:] + jnp.dot(wp_ref[j], p_sl,
                                preferred_element_type=jnp.float32)
    return _combine_dx(u0, b, pp), _combine_dx(u1, b, pp)


def _fused(x_ref, skip_ref, muph_ref, wc1_ref, wm1_ref, wp1_ref, b1_ref,
           wys0_ref, wys1_ref, bs_ref, g1_ref, be1_ref,
           wc2_ref, wm2_ref, wp2_ref, b2_ref, g2_ref, be2_ref,
           mv_ref, gm_ref,
           out_ref,
           mega_ref, act_ref,
           y1s_ref, yss_ref,
           s1a_ref, q1a_ref, s2a_ref, q2a_ref,
           sc1_ref, sh1_ref, sc2_ref, sh2_ref,
           *, stride, margin, pp, hh, width, count):
    ph = pl.program_id(0)
    i = pl.program_id(1)
    nimg = x_ref.shape[0]
    c2, p4 = x_ref.shape[1], x_ref.shape[2]
    d = skip_ref.shape[1]
    m = margin
    mv = mv_ref[...]

    @pl.when(ph == 0)
    def _conv1():
        s = q = None
        for k in range(nimg):
            img = i * nimg + k
            _zero_margins(mega_ref, d + c2 + d, m, pp)

            uph = jnp.dot(x_ref[k].astype(jnp.bfloat16),
                          muph_ref[...], preferred_element_type=jnp.float32)
            mega_ref[d:d + c2, m:m + pp] = uph.astype(jnp.bfloat16)

            sk = skip_ref[k].astype(jnp.bfloat16)
            _fill_rows(mega_ref, 0, sk, hh, width, stride, m, 1, 2)
            _fill_rows(mega_ref, d + c2, sk, hh, width, stride, m, 0, 2)

            raw0, raw1 = _conv9(mega_ref, wc1_ref, wm1_ref, wp1_ref,
                                d + c2, c2 + d, b1_ref[...],
                                stride=stride, m=m, pp=pp, d=d)

            bsv = bs_ref[...]
            t_up = jnp.dot(wys0_ref[...], mega_ref[d:d + c2, m:m + pp],
                           preferred_element_type=jnp.float32) + bsv
            yss_ref[img, 0] = (t_up + jnp.dot(
                wys1_ref[...], mega_ref[d + c2:, m:m + pp],
                preferred_element_type=jnp.float32)).astype(jnp.bfloat16)
            yss_ref[img, 1] = (t_up + jnp.dot(
                wys1_ref[...], mega_ref[:d, m:m + pp],
                preferred_element_type=jnp.float32)).astype(jnp.bfloat16)

            y1s_ref[img, 0] = raw0
            y1s_ref[img, 1] = raw1
            sk_ = (jnp.dot(raw0, mv, preferred_element_type=jnp.float32)
                   + jnp.dot(raw1, mv, preferred_element_type=jnp.float32))
            qk = (jnp.dot(raw0 * raw0, mv, preferred_element_type=jnp.float32)
                  + jnp.dot(raw1 * raw1, mv,
                            preferred_element_type=jnp.float32))
            s = sk_ if s is None else s + sk_
            q = qk if q is None else q + qk
        s1a_ref[...] = jnp.where(i == 0, s, s1a_ref[...] + s)
        q1a_ref[...] = jnp.where(i == 0, q, q1a_ref[...] + q)

    @pl.when((ph == 1) & (i == 0))
    def _bn1():
        sc, sh = _bn_from_sums(s1a_ref, q1a_ref, g1_ref, be1_ref, count)
        sc1_ref[...] = sc
        sh1_ref[...] = sh

    @pl.when(ph == 1)
    def _conv2():
        s = q = None
        sc, sh, gm = sc1_ref[...], sh1_ref[...], gm_ref[...]
        for k in range(nimg):
            img = i * nimg + k
            _zero_margins(act_ref, 2 * d, m, pp)

            act1 = _gelu_exact(y1s_ref[img, 1].astype(jnp.float32) * sc + sh)
            act_ref[0:d, m:m + pp] = act1.astype(jnp.bfloat16) * gm
            act0 = _gelu_exact(y1s_ref[img, 0].astype(jnp.float32) * sc + sh)
            act_ref[d:, m:m + pp] = act0.astype(jnp.bfloat16) * gm

            raw0, raw1 = _conv9(act_ref, wc2_ref, wm2_ref, wp2_ref,
                                d, d, b2_ref[...],
                                stride=stride, m=m, pp=pp, d=d)

            y1s_ref[img, 0] = raw0
            y1s_ref[img, 1] = raw1
            sk_ = (jnp.dot(raw0, mv, preferred_element_type=jnp.float32)
                   + jnp.dot(raw1, mv, preferred_element_type=jnp.float32))
            qk = (jnp.dot(raw0 * raw0, mv, preferred_element_type=jnp.float32)
                  + jnp.dot(raw1 * raw1, mv,
                            preferred_element_type=jnp.float32))
            s = sk_ if s is None else s + sk_
            q = qk if q is None else q + qk
        s2a_ref[...] = jnp.where(i == 0, s, s2a_ref[...] + s)
        q2a_ref[...] = jnp.where(i == 0, q, q2a_ref[...] + q)

    @pl.when((ph == 2) & (i == 0))
    def _bn2():
        sc, sh = _bn_from_sums(s2a_ref, q2a_ref, g2_ref, be2_ref, count)
        sc2_ref[...] = sc
        sh2_ref[...] = sh

    @pl.when(ph == 2)
    def _epilogue():
        sc, sh = sc2_ref[...], sh2_ref[...]
        for k in range(nimg):
            img = i * nimg + k
            for py in range(2):
                act = _gelu_exact(y1s_ref[img, py].astype(jnp.float32)
                                  * sc + sh)
                v = act + yss_ref[img, py].astype(jnp.float32)
                for r in range(hh):
                    fr = 2 * r + py
                    out_ref[k, :, fr * width:(fr + 1) * width] = (
                        v[:, r * stride:r * stride + width])


def kernel(x, skip, w1, b1, g1, be1, w2, b2, g2, be2, wsx, wss, bs):
    n, c2, hh, ww = x.shape
    _, d, hgt, wid = skip.shape
    p4, p = hh * ww, hgt * wid
    stride = wid + 2
    pp = hh * stride
    m = max(128, pl.cdiv(stride + 1, 128) * 128)
    slen = 2 * m + pp
    bf16, f32 = jnp.bfloat16, jnp.float32

    xf = x.reshape(n, c2, p4)
    sf = skip.reshape(n, d, p)

    wu = w1[:, :, :c2]
    wsk = w1[:, :, c2:]

    def taps(w, dy):
        return jnp.stack([w[(dy + 1) * 3 + (dx + 1)] for dx in (-1, 0, 1)])

    wc1 = jnp.concatenate([
        jnp.concatenate([taps(wsk, 1), taps(wu, 0) + taps(wu, 1),
                         taps(wsk, 0)], axis=2),
        jnp.concatenate([taps(wsk, 0), taps(wu, -1) + taps(wu, 0),
                         taps(wsk, -1)], axis=2)], axis=1).astype(bf16)
    wm1 = jnp.concatenate([taps(wsk, -1), taps(wu, -1)], axis=2).astype(bf16)
    wp1 = jnp.concatenate([taps(wu, 1), taps(wsk, 1)], axis=2).astype(bf16)
    wc2 = jnp.concatenate([
        jnp.concatenate([taps(w2, 1), taps(w2, 0)], axis=2),
        jnp.concatenate([taps(w2, 0), taps(w2, -1)], axis=2)],
        axis=1).astype(bf16)
    wm2 = taps(w2, -1).astype(bf16)
    wp2 = taps(w2, 1).astype(bf16)
    wys0 = wsx.astype(bf16)
    wys1 = wss.astype(bf16)

    rr = np.arange(pp) // stride
    cc = np.arange(pp) % stride
    interior = cc < wid
    src = np.where(interior, rr * ww + np.minimum(cc, wid - 1) // 2, -1)
    muph = jnp.asarray(np.arange(p4)[:, None] == src[None, :], bf16)
    maskv = jnp.asarray(interior[:, None], bf16)
    gmask = jnp.asarray(interior[None, :], bf16)

    g1c, be1c = g1.reshape(d, 1), be1.reshape(d, 1)
    g2c, be2c = g2.reshape(d, 1), be2.reshape(d, 1)
    b1b, b2b = b1.astype(bf16), b2.astype(bf16)
    c3s = d + c2 + d

    imgs = next(k for k in (4, 2, 1) if n % k == 0)

    def img0(pidx, ii):
        return (jnp.where(pidx == 0, ii, 0), 0, 0)

    cnst = lambda pidx, ii: (0, 0)
    cnst3 = lambda pidx, ii: (0, 0, 0)

    out = pl.pallas_call(
        partial(_fused, stride=stride, margin=m, pp=pp, hh=hh, width=wid,
                count=float(n * p)),
        grid=(3, n // imgs),
        in_specs=[
            pl.BlockSpec((imgs, c2, p4), img0),
            pl.BlockSpec((imgs, d, p), img0),
            pl.BlockSpec((p4, pp), cnst),
            pl.BlockSpec((3, 2 * d, c3s), cnst3),
            pl.BlockSpec((3, d, d + c2), cnst3),
            pl.BlockSpec((3, d, c2 + d), cnst3),
            pl.BlockSpec((d, 1), cnst),
            pl.BlockSpec((d, c2), cnst),
            pl.BlockSpec((d, d), cnst),
            pl.BlockSpec((d, 1), cnst),
            pl.BlockSpec((d, 1), cnst),
            pl.BlockSpec((d, 1), cnst),
            pl.BlockSpec((3, 2 * d, 2 * d), cnst3),
            pl.BlockSpec((3, d, d), cnst3),
            pl.BlockSpec((3, d, d), cnst3),
            pl.BlockSpec((d, 1), cnst),
            pl.BlockSpec((d, 1), cnst),
            pl.BlockSpec((d, 1), cnst),
            pl.BlockSpec((pp, 1), cnst),
            pl.BlockSpec((1, pp), cnst),
        ],
        out_specs=pl.BlockSpec(
            (imgs, d, p),
            lambda pidx, ii: (jnp.where(pidx == 2, ii, 0), 0, 0)),
        out_shape=jax.ShapeDtypeStruct((n, d, p), f32),
        scratch_shapes=[
            pltpu.VMEM((c3s, slen), bf16),
            pltpu.VMEM((2 * d, slen), bf16),
            pltpu.VMEM((n, 2, d, pp), bf16),
            pltpu.VMEM((n, 2, d, pp), bf16),
            pltpu.VMEM((d, 1), f32),
            pltpu.VMEM((d, 1), f32),
            pltpu.VMEM((d, 1), f32),
            pltpu.VMEM((d, 1), f32),
            pltpu.VMEM((d, 1), f32),
            pltpu.VMEM((d, 1), f32),
            pltpu.VMEM((d, 1), f32),
            pltpu.VMEM((d, 1), f32),
        ],
        compiler_params=pltpu.CompilerParams(
            dimension_semantics=("arbitrary", "arbitrary"),
            vmem_limit_bytes=56 * 1024 * 1024),
    )(xf, sf, muph, wc1, wm1, wp1, b1b, wys0, wys1, bs, g1c, be1c,
      wc2, wm2, wp2, b2b, g2c, be2c, maskv, gmask)

    return out.reshape(n, d, hgt, wid)


# speedup vs baseline: 1.2669x; 1.0024x over previous
from functools import partial

import numpy as np
import jax
import jax.numpy as jnp
from jax import lax
from jax.experimental import pallas as pl
from jax.experimental.pallas import tpu as pltpu

_EPS = 1e-5
_INV_SQRT2 = 0.7071067811865475


def _gelu_exact(v):
    return 0.5 * v * (1.0 + lax.erf(v * _INV_SQRT2))


def _combine_dx(u, b, pp):
    c = u[1].astype(jnp.bfloat16)
    l = u[2].astype(jnp.bfloat16)
    r = u[0].astype(jnp.bfloat16)
    left = jnp.concatenate([l[:, 1:], l[:, :1]], axis=1)
    right = jnp.concatenate([r[:, -1:], r[:, :-1]], axis=1)
    return c + left + right + b


def _zero_margins(slab, rows, m, pp):
    z = jnp.zeros((rows, m), jnp.bfloat16)
    slab[:, 0:m] = z
    slab[:, m + pp:m + pp + m] = z


def _fill_rows(slab, r0, val, nrows, width, stride, m, row0, rstep):
    for i in range(nrows):
        r = row0 + i * rstep
        slab[r0:r0 + val.shape[0],
             m + i * stride:m + i * stride + width] = (
            val[:, r * width:(r + 1) * width])


def _zero_guards(slab, rows, nrows, width, stride, m):
    zg = jnp.zeros((rows, stride - width), jnp.bfloat16)
    for i in range(nrows):
        slab[:, m + i * stride + width:m + (i + 1) * stride] = zg


def _bn_from_sums(s_ref, q_ref, g_ref, be_ref, count):
    tot = s_ref[...]
    totsq = q_ref[...]
    mu = tot / count
    var = totsq / count - mu * mu
    inv = lax.rsqrt(jnp.maximum(var, 0.0) + _EPS)
    sc = g_ref[...] * inv
    sh = be_ref[...] - mu * sc
    return sc, sh


def _conv9(mega, wc_ref, wm_ref, wp_ref, lo_rows, hi_rows, b, *,
           stride, m, pp, d):
    c_sl = mega[:, m:m + pp]
    m_sl = mega[0:lo_rows, m - stride:m - stride + pp]
    p_sl = mega[mega.shape[0] - hi_rows:, m + stride:m + stride + pp]
    u0 = [None, None, None]
    u1 = [None, None, None]
    for j in range(3):
        t = jnp.dot(wc_ref[j], c_sl, preferred_element_type=jnp.float32)
        u0[j] = t[:d] + jnp.dot(wm_ref[j], m_sl,
                                preferred_element_type=jnp.float32)
        u1[j] = t[d:] + jnp.dot(wp_ref[j], p_sl,
                                preferred_element_type=jnp.float32)
    return _combine_dx(u0, b, pp), _combine_dx(u1, b, pp)


def _fused(x_ref, skip_ref, muph_ref, wc1_ref, wm1_ref, wp1_ref, b1_ref,
           wys0_ref, wys1_ref, bs_ref, g1_ref, be1_ref,
           wc2_ref, wm2_ref, wp2_ref, b2_ref, g2_ref, be2_ref,
           mv_ref, gm_ref,
           out_ref,
           mega_ref, act_ref,
           y1s_ref, yss_ref,
           s1a_ref, q1a_ref, s2a_ref, q2a_ref,
           sc1_ref, sh1_ref, sc2_ref, sh2_ref,
           *, stride, margin, pp, hh, width, count):
    ph = pl.program_id(0)
    i = pl.program_id(1)
    nimg = x_ref.shape[0]
    c2, p4 = x_ref.shape[1], x_ref.shape[2]
    d = skip_ref.shape[1]
    m = margin
    mv = mv_ref[...]

    @pl.when((ph == 0) & (i == 0))
    def _init_mega():
        _zero_margins(mega_ref, d + c2 + d, m, pp)
        _zero_guards(mega_ref, d + c2 + d, hh, width, stride, m)

    @pl.when((ph == 1) & (i == 0))
    def _init_act():
        _zero_margins(act_ref, 2 * d, m, pp)

    @pl.when(ph == 0)
    def _conv1():
        s = q = None
        for k in range(nimg):
            img = i * nimg + k

            uph = jnp.dot(x_ref[k].astype(jnp.bfloat16),
                          muph_ref[...], preferred_element_type=jnp.float32)
            mega_ref[d:d + c2, m:m + pp] = uph.astype(jnp.bfloat16)

            sk = skip_ref[k].astype(jnp.bfloat16)
            _fill_rows(mega_ref, 0, sk, hh, width, stride, m, 1, 2)
            _fill_rows(mega_ref, d + c2, sk, hh, width, stride, m, 0, 2)

            raw0, raw1 = _conv9(mega_ref, wc1_ref, wm1_ref, wp1_ref,
                                d + c2, c2 + d, b1_ref[...],
                                stride=stride, m=m, pp=pp, d=d)

            bsv = bs_ref[...]
            t_up = jnp.dot(wys0_ref[...], mega_ref[d:d + c2, m:m + pp],
                           preferred_element_type=jnp.float32) + bsv
            yss_ref[img, 0] = (t_up + jnp.dot(
                wys1_ref[...], mega_ref[d + c2:, m:m + pp],
                preferred_element_type=jnp.float32)).astype(jnp.bfloat16)
            yss_ref[img, 1] = (t_up + jnp.dot(
                wys1_ref[...], mega_ref[:d, m:m + pp],
                preferred_element_type=jnp.float32)).astype(jnp.bfloat16)

            y1s_ref[img, 0] = raw0
            y1s_ref[img, 1] = raw1
            sk_ = (jnp.dot(raw0, mv, preferred_element_type=jnp.float32)
                   + jnp.dot(raw1, mv, preferred_element_type=jnp.float32))
            qk = (jnp.dot(raw0 * raw0, mv, preferred_element_type=jnp.float32)
                  + jnp.dot(raw1 * raw1, mv,
                            preferred_element_type=jnp.float32))
            s = sk_ if s is None else s + sk_
            q = qk if q is None else q + qk
        s1a_ref[...] = jnp.where(i == 0, s, s1a_ref[...] + s)
        q1a_ref[...] = jnp.where(i == 0, q, q1a_ref[...] + q)

    @pl.when((ph == 1) & (i == 0))
    def _bn1():
        sc, sh = _bn_from_sums(s1a_ref, q1a_ref, g1_ref, be1_ref, count)
        sc1_ref[...] = sc
        sh1_ref[...] = sh

    @pl.when(ph == 1)
    def _conv2():
        s = q = None
        sc, sh, gm = sc1_ref[...], sh1_ref[...], gm_ref[...]
        for k in range(nimg):
            img = i * nimg + k

            act1 = _gelu_exact(y1s_ref[img, 1].astype(jnp.float32) * sc + sh)
            act_ref[0:d, m:m + pp] = act1.astype(jnp.bfloat16) * gm
            act0 = _gelu_exact(y1s_ref[img, 0].astype(jnp.float32) * sc + sh)
            act_ref[d:, m:m + pp] = act0.astype(jnp.bfloat16) * gm

            raw0, raw1 = _conv9(act_ref, wc2_ref, wm2_ref, wp2_ref,
                                d, d, b2_ref[...],
                                stride=stride, m=m, pp=pp, d=d)

            y1s_ref[img, 0] = raw0
            y1s_ref[img, 1] = raw1
            sk_ = (jnp.dot(raw0, mv, preferred_element_type=jnp.float32)
                   + jnp.dot(raw1, mv, preferred_element_type=jnp.float32))
            qk = (jnp.dot(raw0 * raw0, mv, preferred_element_type=jnp.float32)
                  + jnp.dot(raw1 * raw1, mv,
                            preferred_element_type=jnp.float32))
            s = sk_ if s is None else s + sk_
            q = qk if q is None else q + qk
        s2a_ref[...] = jnp.where(i == 0, s, s2a_ref[...] + s)
        q2a_ref[...] = jnp.where(i == 0, q, q2a_ref[...] + q)

    @pl.when((ph == 2) & (i == 0))
    def _bn2():
        sc, sh = _bn_from_sums(s2a_ref, q2a_ref, g2_ref, be2_ref, count)
        sc2_ref[...] = sc
        sh2_ref[...] = sh

    @pl.when(ph == 2)
    def _epilogue():
        sc, sh = sc2_ref[...], sh2_ref[...]
        for k in range(nimg):
            img = i * nimg + k
            for py in range(2):
                act = _gelu_exact(y1s_ref[img, py].astype(jnp.float32)
                                  * sc + sh)
                v = act + yss_ref[img, py].astype(jnp.float32)
                for r in range(hh):
                    fr = 2 * r + py
                    out_ref[k, :, fr * width:(fr + 1) * width] = (
                        v[:, r * stride:r * stride + width])


def kernel(x, skip, w1, b1, g1, be1, w2, b2, g2, be2, wsx, wss, bs):
    n, c2, hh, ww = x.shape
    _, d, hgt, wid = skip.shape
    p4, p = hh * ww, hgt * wid
    stride = wid + 2
    pp = hh * stride
    m = max(128, pl.cdiv(stride + 1, 128) * 128)
    slen = 2 * m + pp
    bf16, f32 = jnp.bfloat16, jnp.float32

    xf = x.reshape(n, c2, p4)
    sf = skip.reshape(n, d, p)

    wu = w1[:, :, :c2]
    wsk = w1[:, :, c2:]

    def taps(w, dy):
        return jnp.stack([w[(dy + 1) * 3 + (dx + 1)] for dx in (-1, 0, 1)])

    wc1 = jnp.concatenate([
        jnp.concatenate([taps(wsk, 1), taps(wu, 0) + taps(wu, 1),
                         taps(wsk, 0)], axis=2),
        jnp.concatenate([taps(wsk, 0), taps(wu, -1) + taps(wu, 0),
                         taps(wsk, -1)], axis=2)], axis=1).astype(bf16)
    wm1 = jnp.concatenate([taps(wsk, -1), taps(wu, -1)], axis=2).astype(bf16)
    wp1 = jnp.concatenate([taps(wu, 1), taps(wsk, 1)], axis=2).astype(bf16)
    wc2 = jnp.concatenate([
        jnp.concatenate([taps(w2, 1), taps(w2, 0)], axis=2),
        jnp.concatenate([taps(w2, 0), taps(w2, -1)], axis=2)],
        axis=1).astype(bf16)
    wm2 = taps(w2, -1).astype(bf16)
    wp2 = taps(w2, 1).astype(bf16)
    wys0 = wsx.astype(bf16)
    wys1 = wss.astype(bf16)

    rr = np.arange(pp) // stride
    cc = np.arange(pp) % stride
    interior = cc < wid
    src = np.where(interior, rr * ww + np.minimum(cc, wid - 1) // 2, -1)
    muph = jnp.asarray(np.arange(p4)[:, None] == src[None, :], bf16)
    maskv = jnp.asarray(interior[:, None], bf16)
    gmask = jnp.asarray(interior[None, :], bf16)

    g1c, be1c = g1.reshape(d, 1), be1.reshape(d, 1)
    g2c, be2c = g2.reshape(d, 1), be2.reshape(d, 1)
    b1b, b2b = b1.astype(bf16), b2.astype(bf16)
    c3s = d + c2 + d

    imgs = next(k for k in (4, 2, 1) if n % k == 0)

    def img0(pidx, ii):
        return (jnp.where(pidx == 0, ii, 0), 0, 0)

    cnst = lambda pidx, ii: (0, 0)
    cnst3 = lambda pidx, ii: (0, 0, 0)

    out = pl.pallas_call(
        partial(_fused, stride=stride, margin=m, pp=pp, hh=hh, width=wid,
                count=float(n * p)),
        grid=(3, n // imgs),
        in_specs=[
            pl.BlockSpec((imgs, c2, p4), img0),
            pl.BlockSpec((imgs, d, p), img0),
            pl.BlockSpec((p4, pp), cnst),
            pl.BlockSpec((3, 2 * d, c3s), cnst3),
            pl.BlockSpec((3, d, d + c2), cnst3),
            pl.BlockSpec((3, d, c2 + d), cnst3),
            pl.BlockSpec((d, 1), cnst),
            pl.BlockSpec((d, c2), cnst),
            pl.BlockSpec((d, d), cnst),
            pl.BlockSpec((d, 1), cnst),
            pl.BlockSpec((d, 1), cnst),
            pl.BlockSpec((d, 1), cnst),
            pl.BlockSpec((3, 2 * d, 2 * d), cnst3),
            pl.BlockSpec((3, d, d), cnst3),
            pl.BlockSpec((3, d, d), cnst3),
            pl.BlockSpec((d, 1), cnst),
            pl.BlockSpec((d, 1), cnst),
            pl.BlockSpec((d, 1), cnst),
            pl.BlockSpec((pp, 1), cnst),
            pl.BlockSpec((1, pp), cnst),
        ],
        out_specs=pl.BlockSpec(
            (imgs, d, p),
            lambda pidx, ii: (jnp.where(pidx == 2, ii, 0), 0, 0)),
        out_shape=jax.ShapeDtypeStruct((n, d, p), f32),
        scratch_shapes=[
            pltpu.VMEM((c3s, slen), bf16),
            pltpu.VMEM((2 * d, slen), bf16),
            pltpu.VMEM((n, 2, d, pp), bf16),
            pltpu.VMEM((n, 2, d, pp), bf16),
            pltpu.VMEM((d, 1), f32),
            pltpu.VMEM((d, 1), f32),
            pltpu.VMEM((d, 1), f32),
            pltpu.VMEM((d, 1), f32),
            pltpu.VMEM((d, 1), f32),
            pltpu.VMEM((d, 1), f32),
            pltpu.VMEM((d, 1), f32),
            pltpu.VMEM((d, 1), f32),
        ],
        compiler_params=pltpu.CompilerParams(
            dimension_semantics=("arbitrary", "arbitrary"),
            vmem_limit_bytes=56 * 1024 * 1024),
    )(xf, sf, muph, wc1, wm1, wp1, b1b, wys0, wys1, bs, g1c, be1c,
      wc2, wm2, wp2, b2b, g2c, be2c, maskv, gmask)

    return out.reshape(n, d, hgt, wid)


# speedup vs baseline: 1.2676x; 1.0005x over previous
from functools import partial

import numpy as np
import jax
import jax.numpy as jnp
from jax import lax
from jax.experimental import pallas as pl
from jax.experimental.pallas import tpu as pltpu

_EPS = 1e-5
_INV_SQRT2 = 0.7071067811865475


def _gelu_exact(v):
    return 0.5 * v * (1.0 + lax.erf(v * _INV_SQRT2))


def _combine_dx(u, b, pp):
    c = u[1].astype(jnp.bfloat16)
    l = u[2].astype(jnp.bfloat16)
    r = u[0].astype(jnp.bfloat16)
    left = jnp.concatenate([l[:, 1:], l[:, :1]], axis=1)
    right = jnp.concatenate([r[:, -1:], r[:, :-1]], axis=1)
    return c + left + right + b


def _zero_margins(slab, rows, m, pp):
    z = jnp.zeros((rows, m), jnp.bfloat16)
    slab[:, 0:m] = z
    slab[:, m + pp:m + pp + m] = z


def _fill_rows(slab, r0, val, nrows, width, stride, m, row0, rstep):
    for i in range(nrows):
        r = row0 + i * rstep
        slab[r0:r0 + val.shape[0],
             m + i * stride:m + i * stride + width] = (
            val[:, r * width:(r + 1) * width])


def _zero_guards(slab, rows, nrows, width, stride, m):
    zg = jnp.zeros((rows, stride - width), jnp.bfloat16)
    for i in range(nrows):
        slab[:, m + i * stride + width:m + (i + 1) * stride] = zg


def _bn_from_sums(s_ref, q_ref, g_ref, be_ref, count):
    tot = s_ref[...]
    totsq = q_ref[...]
    mu = tot / count
    var = totsq / count - mu * mu
    inv = lax.rsqrt(jnp.maximum(var, 0.0) + _EPS)
    sc = g_ref[...] * inv
    sh = be_ref[...] - mu * sc
    return sc, sh


def _conv9(mega, wc_ref, wm_ref, wp_ref, lo_rows, hi_rows, b, *,
           stride, m, pp, d):
    c_sl = mega[:, m:m + pp]
    m_sl = mega[0:lo_rows, m - stride:m - stride + pp]
    p_sl = mega[mega.shape[0] - hi_rows:, m + stride:m + stride + pp]
    u0 = [None, None, None]
    u1 = [None, None, None]
    for j in range(3):
        t = jnp.dot(wc_ref[j], c_sl, preferred_element_type=jnp.float32)
        u0[j] = t[:d] + jnp.dot(wm_ref[j], m_sl,
                                preferred_element_type=jnp.float32)
        u1[j] = t[d:] + jnp.dot(wp_ref[j], p_sl,
                                preferred_element_type=jnp.float32)
    return _combine_dx(u0, b, pp), _combine_dx(u1, b, pp)


def _fused(x_ref, skip_ref, muph_ref, wc1_ref, wm1_ref, wp1_ref, b1_ref,
           wys0_ref, wys1_ref, bs_ref, g1_ref, be1_ref,
           wc2_ref, wm2_ref, wp2_ref, b2_ref, g2_ref, be2_ref,
           mv_ref, gm_ref,
           out_ref,
           mega_ref, act_ref,
           y1s_ref, yss_ref,
           s1a_ref, q1a_ref, s2a_ref, q2a_ref,
           sc1_ref, sh1_ref, sc2_ref, sh2_ref,
           *, stride, margin, pp, hh, width, count):
    ph = pl.program_id(0)
    i = pl.program_id(1)
    nimg = x_ref.shape[0]
    c2, p4 = x_ref.shape[1], x_ref.shape[2]
    d = skip_ref.shape[1]
    m = margin
    mv = mv_ref[...]

    @pl.when((ph == 0) & (i == 0))
    def _init_mega():
        _zero_margins(mega_ref, d + c2 + d, m, pp)
        _zero_guards(mega_ref, d + c2 + d, hh, width, stride, m)

    @pl.when((ph == 1) & (i == 0))
    def _init_act():
        _zero_margins(act_ref, 2 * d, m, pp)

    @pl.when(ph == 0)
    def _conv1():
        s = q = None
        for k in range(nimg):
            img = i * nimg + k

            uph = jnp.dot(x_ref[k].astype(jnp.bfloat16),
                          muph_ref[...], preferred_element_type=jnp.float32)
            mega_ref[d:d + c2, m:m + pp] = uph.astype(jnp.bfloat16)

            sk = skip_ref[k].astype(jnp.bfloat16)
            _fill_rows(mega_ref, 0, sk, hh, width, stride, m, 1, 2)
            _fill_rows(mega_ref, d + c2, sk, hh, width, stride, m, 0, 2)

            raw0, raw1 = _conv9(mega_ref, wc1_ref, wm1_ref, wp1_ref,
                                d + c2, c2 + d, b1_ref[...],
                                stride=stride, m=m, pp=pp, d=d)

            bsv = bs_ref[...]
            t_up = jnp.dot(wys0_ref[...], mega_ref[d:d + c2, m:m + pp],
                           preferred_element_type=jnp.float32) + bsv
            yss_ref[img, 0] = (t_up + jnp.dot(
                wys1_ref[...], mega_ref[d + c2:, m:m + pp],
                preferred_element_type=jnp.float32)).astype(jnp.bfloat16)
            yss_ref[img, 1] = (t_up + jnp.dot(
                wys1_ref[...], mega_ref[:d, m:m + pp],
                preferred_element_type=jnp.float32)).astype(jnp.bfloat16)

            y1s_ref[img, 0] = raw0
            y1s_ref[img, 1] = raw1
            sk_ = (jnp.dot(raw0, mv, preferred_element_type=jnp.float32)
                   + jnp.dot(raw1, mv, preferred_element_type=jnp.float32))
            qk = (jnp.dot(raw0 * raw0, mv, preferred_element_type=jnp.float32)
                  + jnp.dot(raw1 * raw1, mv,
                            preferred_element_type=jnp.float32))
            s = sk_ if s is None else s + sk_
            q = qk if q is None else q + qk
        s1a_ref[...] = jnp.where(i == 0, s, s1a_ref[...] + s)
        q1a_ref[...] = jnp.where(i == 0, q, q1a_ref[...] + q)

    @pl.when((ph == 1) & (i == 0))
    def _bn1():
        sc, sh = _bn_from_sums(s1a_ref, q1a_ref, g1_ref, be1_ref, count)
        sc1_ref[...] = sc
        sh1_ref[...] = sh

    @pl.when(ph == 1)
    def _conv2():
        s = q = None
        sc, sh, gm = sc1_ref[...], sh1_ref[...], gm_ref[...]
        for k in range(nimg):
            img = i * nimg + k

            act1 = _gelu_exact(y1s_ref[img, 1].astype(jnp.float32) * sc + sh)
            act_ref[0:d, m:m + pp] = act1.astype(jnp.bfloat16) * gm
            act0 = _gelu_exact(y1s_ref[img, 0].astype(jnp.float32) * sc + sh)
            act_ref[d:, m:m + pp] = act0.astype(jnp.bfloat16) * gm

            raw0, raw1 = _conv9(act_ref, wc2_ref, wm2_ref, wp2_ref,
                                d, d, b2_ref[...],
                                stride=stride, m=m, pp=pp, d=d)

            y1s_ref[img, 0] = raw0
            y1s_ref[img, 1] = raw1
            sk_ = (jnp.dot(raw0, mv, preferred_element_type=jnp.float32)
                   + jnp.dot(raw1, mv, preferred_element_type=jnp.float32))
            qk = (jnp.dot(raw0 * raw0, mv, preferred_element_type=jnp.float32)
                  + jnp.dot(raw1 * raw1, mv,
                            preferred_element_type=jnp.float32))
            s = sk_ if s is None else s + sk_
            q = qk if q is None else q + qk
        s2a_ref[...] = jnp.where(i == 0, s, s2a_ref[...] + s)
        q2a_ref[...] = jnp.where(i == 0, q, q2a_ref[...] + q)

    @pl.when((ph == 2) & (i == 0))
    def _bn2():
        sc, sh = _bn_from_sums(s2a_ref, q2a_ref, g2_ref, be2_ref, count)
        sc2_ref[...] = sc
        sh2_ref[...] = sh

    @pl.when(ph == 2)
    def _epilogue():
        sc, sh = sc2_ref[...], sh2_ref[...]
        for k in range(nimg):
            img = i * nimg + k
            for py in range(2):
                act = _gelu_exact(y1s_ref[img, py].astype(jnp.float32)
                                  * sc + sh)
                v = act + yss_ref[img, py].astype(jnp.float32)
                for r in range(hh):
                    fr = 2 * r + py
                    out_ref[k, :, fr * width:(fr + 1) * width] = (
                        v[:, r * stride:r * stride + width])


def kernel(x, skip, w1, b1, g1, be1, w2, b2, g2, be2, wsx, wss, bs):
    n, c2, hh, ww = x.shape
    _, d, hgt, wid = skip.shape
    p4, p = hh * ww, hgt * wid
    stride = wid + 2
    pp = hh * stride
    m = max(128, pl.cdiv(stride + 1, 128) * 128)
    slen = 2 * m + pp
    bf16, f32 = jnp.bfloat16, jnp.float32

    xf = x.reshape(n, c2, p4)
    sf = skip.reshape(n, d, p)

    wu = w1[:, :, :c2]
    wsk = w1[:, :, c2:]

    def taps(w, dy):
        return jnp.stack([w[(dy + 1) * 3 + (dx + 1)] for dx in (-1, 0, 1)])

    wc1 = jnp.concatenate([
        jnp.concatenate([taps(wsk, 1), taps(wu, 0) + taps(wu, 1),
                         taps(wsk, 0)], axis=2),
        jnp.concatenate([taps(wsk, 0), taps(wu, -1) + taps(wu, 0),
                         taps(wsk, -1)], axis=2)], axis=1).astype(bf16)
    wm1 = jnp.concatenate([taps(wsk, -1), taps(wu, -1)], axis=2).astype(bf16)
    wp1 = jnp.concatenate([taps(wu, 1), taps(wsk, 1)], axis=2).astype(bf16)
    wc2 = jnp.concatenate([
        jnp.concatenate([taps(w2, 1), taps(w2, 0)], axis=2),
        jnp.concatenate([taps(w2, 0), taps(w2, -1)], axis=2)],
        axis=1).astype(bf16)
    wm2 = taps(w2, -1).astype(bf16)
    wp2 = taps(w2, 1).astype(bf16)
    wys0 = wsx.astype(bf16)
    wys1 = wss.astype(bf16)

    rr = np.arange(pp) // stride
    cc = np.arange(pp) % stride
    interior = cc < wid
    src = np.where(interior, rr * ww + np.minimum(cc, wid - 1) // 2, -1)
    muph = jnp.asarray(np.arange(p4)[:, None] == src[None, :], bf16)
    maskv = jnp.asarray(interior[:, None], bf16)
    gmask = jnp.asarray(interior[None, :], bf16)

    g1c, be1c = g1.reshape(d, 1), be1.reshape(d, 1)
    g2c, be2c = g2.reshape(d, 1), be2.reshape(d, 1)
    b1b, b2b = b1.astype(bf16), b2.astype(bf16)
    c3s = d + c2 + d

    imgs = next(k for k in (4, 2, 1) if n % k == 0)

    def img0(pidx, ii):
        return (jnp.where(pidx == 0, ii, 0), 0, 0)

    cnst = lambda pidx, ii: (0, 0)
    cnst3 = lambda pidx, ii: (0, 0, 0)

    out = pl.pallas_call(
        partial(_fused, stride=stride, margin=m, pp=pp, hh=hh, width=wid,
                count=float(n * p)),
        grid=(3, n // imgs),
        in_specs=[
            pl.BlockSpec((imgs, c2, p4), img0),
            pl.BlockSpec((imgs, d, p), img0),
            pl.BlockSpec((p4, pp), cnst),
            pl.BlockSpec((3, 2 * d, c3s), cnst3),
            pl.BlockSpec((3, d, d + c2), cnst3),
            pl.BlockSpec((3, d, c2 + d), cnst3),
            pl.BlockSpec((d, 1), cnst),
            pl.BlockSpec((d, c2), cnst),
            pl.BlockSpec((d, d), cnst),
            pl.BlockSpec((d, 1), cnst),
            pl.BlockSpec((d, 1), cnst),
            pl.BlockSpec((d, 1), cnst),
            pl.BlockSpec((3, 2 * d, 2 * d), cnst3),
            pl.BlockSpec((3, d, d), cnst3),
            pl.BlockSpec((3, d, d), cnst3),
            pl.BlockSpec((d, 1), cnst),
            pl.BlockSpec((d, 1), cnst),
            pl.BlockSpec((d, 1), cnst),
            pl.BlockSpec((pp, 1), cnst),
            pl.BlockSpec((1, pp), cnst),
        ],
        out_specs=pl.BlockSpec(
            (imgs, d, p),
            lambda pidx, ii: (jnp.where(pidx == 2, ii, 0), 0, 0)),
        out_shape=jax.ShapeDtypeStruct((n, d, p), f32),
        scratch_shapes=[
            pltpu.VMEM((c3s, slen), bf16),
            pltpu.VMEM((2 * d, slen), bf16),
            pltpu.VMEM((n, 2, d, pp), bf16),
            pltpu.VMEM((n, 2, d, pp), bf16),
            pltpu.VMEM((d, 1), f32),
            pltpu.VMEM((d, 1), f32),
            pltpu.VMEM((d, 1), f32),
            pltpu.VMEM((d, 1), f32),
            pltpu.VMEM((d, 1), f32),
            pltpu.VMEM((d, 1), f32),
            pltpu.VMEM((d, 1), f32),
            pltpu.VMEM((d, 1), f32),
        ],
        compiler_params=pltpu.CompilerParams(
            dimension_semantics=("arbitrary", "arbitrary"),
            vmem_limit_bytes=61_000_000),
    )(xf, sf, muph, wc1, wm1, wp1, b1b, wys0, wys1, bs, g1c, be1c,
      wc2, wm2, wp2, b2b, g2c, be2c, maskv, gmask)

    return out.reshape(n, d, hgt, wid)


# speedup vs baseline: 1.2781x; 1.0083x over previous
from functools import partial

import numpy as np
import jax
import jax.numpy as jnp
from jax import lax
from jax.experimental import pallas as pl
from jax.experimental.pallas import tpu as pltpu

_EPS = 1e-5
_INV_SQRT2 = 0.7071067811865475


def _gelu_exact(v):
    return 0.5 * v * (1.0 + lax.erf(v * _INV_SQRT2))


def _combine_dx(u, b, pp):
    c = u[1].astype(jnp.bfloat16)
    l = u[2].astype(jnp.bfloat16)
    r = u[0].astype(jnp.bfloat16)
    left = jnp.concatenate([l[:, 1:], l[:, :1]], axis=1)
    right = jnp.concatenate([r[:, -1:], r[:, :-1]], axis=1)
    return c + left + right + b


def _zero_margins(slab, rows, m, pp):
    z = jnp.zeros((rows, m), jnp.bfloat16)
    slab[:, 0:m] = z
    slab[:, m + pp:m + pp + m] = z


def _fill_rows(slab, r0, val, nrows, width, stride, m, row0, rstep):
    for i in range(nrows):
        r = row0 + i * rstep
        slab[r0:r0 + val.shape[0],
             m + i * stride:m + i * stride + width] = (
            val[:, r * width:(r + 1) * width])


def _zero_guards(slab, rows, nrows, width, stride, m):
    zg = jnp.zeros((rows, stride - width), jnp.bfloat16)
    for i in range(nrows):
        slab[:, m + i * stride + width:m + (i + 1) * stride] = zg


def _bn_from_sums(s_ref, q_ref, g_ref, be_ref, count):
    tot = s_ref[...]
    totsq = q_ref[...]
    mu = tot / count
    var = totsq / count - mu * mu
    inv = lax.rsqrt(jnp.maximum(var, 0.0) + _EPS)
    sc = g_ref[...] * inv
    sh = be_ref[...] - mu * sc
    return sc, sh


def _conv9(mega, wc_ref, wm_ref, wp_ref, lo_rows, hi_rows, b, *,
           stride, m, pp, d, center=None):
    c_sl = mega[:, m:m + pp] if center is None else center
    m_sl = mega[0:lo_rows, m - stride:m - stride + pp]
    p_sl = mega[mega.shape[0] - hi_rows:, m + stride:m + stride + pp]
    u0 = [None, None, None]
    u1 = [None, None, None]
    for j in range(3):
        t = jnp.dot(wc_ref[j], c_sl, preferred_element_type=jnp.float32)
        u0[j] = t[:d] + jnp.dot(wm_ref[j], m_sl,
                                preferred_element_type=jnp.float32)
        u1[j] = t[d:] + jnp.dot(wp_ref[j], p_sl,
                                preferred_element_type=jnp.float32)
    return _combine_dx(u0, b, pp), _combine_dx(u1, b, pp)


def _fused(x_ref, skip_ref, muph_ref, wc1_ref, wm1_ref, wp1_ref, b1_ref,
           wys0_ref, wys1_ref, bs_ref, g1_ref, be1_ref,
           wc2_ref, wm2_ref, wp2_ref, b2_ref, g2_ref, be2_ref,
           mv_ref, gm_ref,
           out_ref,
           mega_ref, act_ref,
           y1s_ref, yss_ref,
           s1a_ref, q1a_ref, s2a_ref, q2a_ref,
           sc1_ref, sh1_ref, sc2_ref, sh2_ref,
           *, stride, margin, pp, hh, width, count):
    ph = pl.program_id(0)
    i = pl.program_id(1)
    nimg = x_ref.shape[0]
    c2, p4 = x_ref.shape[1], x_ref.shape[2]
    d = skip_ref.shape[1]
    m = margin
    mv = mv_ref[...]

    @pl.when((ph == 0) & (i == 0))
    def _init_mega():
        _zero_margins(mega_ref, d + c2 + d, m, pp)
        _zero_guards(mega_ref, d + c2 + d, hh, width, stride, m)

    @pl.when((ph == 1) & (i == 0))
    def _init_act():
        _zero_margins(act_ref, 2 * d, m, pp)

    @pl.when(ph == 0)
    def _conv1():
        s = q = None
        for k in range(nimg):
            img = i * nimg + k

            uph = jnp.dot(x_ref[k].astype(jnp.bfloat16),
                          muph_ref[...], preferred_element_type=jnp.float32)
            mega_ref[d:d + c2, m:m + pp] = uph.astype(jnp.bfloat16)

            sk = skip_ref[k].astype(jnp.bfloat16)
            _fill_rows(mega_ref, 0, sk, hh, width, stride, m, 1, 2)
            _fill_rows(mega_ref, d + c2, sk, hh, width, stride, m, 0, 2)

            raw0, raw1 = _conv9(mega_ref, wc1_ref, wm1_ref, wp1_ref,
                                d + c2, c2 + d, b1_ref[...],
                                stride=stride, m=m, pp=pp, d=d)

            bsv = bs_ref[...]
            t_up = jnp.dot(wys0_ref[...], mega_ref[d:d + c2, m:m + pp],
                           preferred_element_type=jnp.float32) + bsv
            yss_ref[img, 0] = (t_up + jnp.dot(
                wys1_ref[...], mega_ref[d + c2:, m:m + pp],
                preferred_element_type=jnp.float32)).astype(jnp.bfloat16)
            yss_ref[img, 1] = (t_up + jnp.dot(
                wys1_ref[...], mega_ref[:d, m:m + pp],
                preferred_element_type=jnp.float32)).astype(jnp.bfloat16)

            y1s_ref[img, 0] = raw0
            y1s_ref[img, 1] = raw1
            rc = jnp.concatenate([raw0, raw1], axis=0)
            ts = jnp.dot(rc, mv, preferred_element_type=jnp.float32)
            tq = jnp.dot(rc * rc, mv, preferred_element_type=jnp.float32)
            sk_ = ts[:d] + ts[d:]
            qk = tq[:d] + tq[d:]
            s = sk_ if s is None else s + sk_
            q = qk if q is None else q + qk
        s1a_ref[...] = jnp.where(i == 0, s, s1a_ref[...] + s)
        q1a_ref[...] = jnp.where(i == 0, q, q1a_ref[...] + q)

    @pl.when((ph == 1) & (i == 0))
    def _bn1():
        sc, sh = _bn_from_sums(s1a_ref, q1a_ref, g1_ref, be1_ref, count)
        sc1_ref[...] = sc
        sh1_ref[...] = sh

    @pl.when(ph == 1)
    def _conv2():
        s = q = None
        sc, sh, gm = sc1_ref[...], sh1_ref[...], gm_ref[...]
        for k in range(nimg):
            img = i * nimg + k

            act1 = _gelu_exact(y1s_ref[img, 1].astype(jnp.float32)
                               * sc + sh).astype(jnp.bfloat16) * gm
            act_ref[0:d, m:m + pp] = act1
            act0 = _gelu_exact(y1s_ref[img, 0].astype(jnp.float32)
                               * sc + sh).astype(jnp.bfloat16) * gm
            act_ref[d:, m:m + pp] = act0

            acat = jnp.concatenate([act1, act0], axis=0)
            raw0, raw1 = _conv9(act_ref, wc2_ref, wm2_ref, wp2_ref,
                                d, d, b2_ref[...],
                                stride=stride, m=m, pp=pp, d=d, center=acat)

            y1s_ref[img, 0] = raw0
            y1s_ref[img, 1] = raw1
            rc = jnp.concatenate([raw0, raw1], axis=0)
            ts = jnp.dot(rc, mv, preferred_element_type=jnp.float32)
            tq = jnp.dot(rc * rc, mv, preferred_element_type=jnp.float32)
            sk_ = ts[:d] + ts[d:]
            qk = tq[:d] + tq[d:]
            s = sk_ if s is None else s + sk_
            q = qk if q is None else q + qk
        s2a_ref[...] = jnp.where(i == 0, s, s2a_ref[...] + s)
        q2a_ref[...] = jnp.where(i == 0, q, q2a_ref[...] + q)

    @pl.when((ph == 2) & (i == 0))
    def _bn2():
        sc, sh = _bn_from_sums(s2a_ref, q2a_ref, g2_ref, be2_ref, count)
        sc2_ref[...] = sc
        sh2_ref[...] = sh

    @pl.when(ph == 2)
    def _epilogue():
        sc, sh = sc2_ref[...], sh2_ref[...]
        for k in range(nimg):
            img = i * nimg + k
            for py in range(2):
                act = _gelu_exact(y1s_ref[img, py].astype(jnp.float32)
                                  * sc + sh)
                v = act + yss_ref[img, py].astype(jnp.float32)
                for r in range(hh):
                    fr = 2 * r + py
                    out_ref[k, :, fr * width:(fr + 1) * width] = (
                        v[:, r * stride:r * stride + width])


def kernel(x, skip, w1, b1, g1, be1, w2, b2, g2, be2, wsx, wss, bs):
    n, c2, hh, ww = x.shape
    _, d, hgt, wid = skip.shape
    p4, p = hh * ww, hgt * wid
    stride = wid + 2
    pp = hh * stride
    m = max(128, pl.cdiv(stride + 1, 128) * 128)
    slen = 2 * m + pp
    bf16, f32 = jnp.bfloat16, jnp.float32

    xf = x.reshape(n, c2, p4)
    sf = skip.reshape(n, d, p)

    wu = w1[:, :, :c2]
    wsk = w1[:, :, c2:]

    def taps(w, dy):
        return jnp.stack([w[(dy + 1) * 3 + (dx + 1)] for dx in (-1, 0, 1)])

    wc1 = jnp.concatenate([
        jnp.concatenate([taps(wsk, 1), taps(wu, 0) + taps(wu, 1),
                         taps(wsk, 0)], axis=2),
        jnp.concatenate([taps(wsk, 0), taps(wu, -1) + taps(wu, 0),
                         taps(wsk, -1)], axis=2)], axis=1).astype(bf16)
    wm1 = jnp.concatenate([taps(wsk, -1), taps(wu, -1)], axis=2).astype(bf16)
    wp1 = jnp.concatenate([taps(wu, 1), taps(wsk, 1)], axis=2).astype(bf16)
    wc2 = jnp.concatenate([
        jnp.concatenate([taps(w2, 1), taps(w2, 0)], axis=2),
        jnp.concatenate([taps(w2, 0), taps(w2, -1)], axis=2)],
        axis=1).astype(bf16)
    wm2 = taps(w2, -1).astype(bf16)
    wp2 = taps(w2, 1).astype(bf16)
    wys0 = wsx.astype(bf16)
    wys1 = wss.astype(bf16)

    rr = np.arange(pp) // stride
    cc = np.arange(pp) % stride
    interior = cc < wid
    src = np.where(interior, rr * ww + np.minimum(cc, wid - 1) // 2, -1)
    muph = jnp.asarray(np.arange(p4)[:, None] == src[None, :], bf16)
    maskv = jnp.asarray(interior[:, None], bf16)
    gmask = jnp.asarray(interior[None, :], bf16)

    g1c, be1c = g1.reshape(d, 1), be1.reshape(d, 1)
    g2c, be2c = g2.reshape(d, 1), be2.reshape(d, 1)
    b1b, b2b = b1.astype(bf16), b2.astype(bf16)
    c3s = d + c2 + d

    imgs = next(k for k in (4, 2, 1) if n % k == 0)

    def img0(pidx, ii):
        return (jnp.where(pidx == 0, ii, 0), 0, 0)

    cnst = lambda pidx, ii: (0, 0)
    cnst3 = lambda pidx, ii: (0, 0, 0)

    out = pl.pallas_call(
        partial(_fused, stride=stride, margin=m, pp=pp, hh=hh, width=wid,
                count=float(n * p)),
        grid=(3, n // imgs),
        in_specs=[
            pl.BlockSpec((imgs, c2, p4), img0),
            pl.BlockSpec((imgs, d, p), img0),
            pl.BlockSpec((p4, pp), cnst),
            pl.BlockSpec((3, 2 * d, c3s), cnst3),
            pl.BlockSpec((3, d, d + c2), cnst3),
            pl.BlockSpec((3, d, c2 + d), cnst3),
            pl.BlockSpec((d, 1), cnst),
            pl.BlockSpec((d, c2), cnst),
            pl.BlockSpec((d, d), cnst),
            pl.BlockSpec((d, 1), cnst),
            pl.BlockSpec((d, 1), cnst),
            pl.BlockSpec((d, 1), cnst),
            pl.BlockSpec((3, 2 * d, 2 * d), cnst3),
            pl.BlockSpec((3, d, d), cnst3),
            pl.BlockSpec((3, d, d), cnst3),
            pl.BlockSpec((d, 1), cnst),
            pl.BlockSpec((d, 1), cnst),
            pl.BlockSpec((d, 1), cnst),
            pl.BlockSpec((pp, 1), cnst),
            pl.BlockSpec((1, pp), cnst),
        ],
        out_specs=pl.BlockSpec(
            (imgs, d, p),
            lambda pidx, ii: (jnp.where(pidx == 2, ii, 0), 0, 0)),
        out_shape=jax.ShapeDtypeStruct((n, d, p), f32),
        scratch_shapes=[
            pltpu.VMEM((c3s, slen), bf16),
            pltpu.VMEM((2 * d, slen), bf16),
            pltpu.VMEM((n, 2, d, pp), bf16),
            pltpu.VMEM((n, 2, d, pp), bf16),
            pltpu.VMEM((d, 1), f32),
            pltpu.VMEM((d, 1), f32),
            pltpu.VMEM((d, 1), f32),
            pltpu.VMEM((d, 1), f32),
            pltpu.VMEM((d, 1), f32),
            pltpu.VMEM((d, 1), f32),
            pltpu.VMEM((d, 1), f32),
            pltpu.VMEM((d, 1), f32),
        ],
        compiler_params=pltpu.CompilerParams(
            dimension_semantics=("arbitrary", "arbitrary"),
            vmem_limit_bytes=61_000_000),
    )(xf, sf, muph, wc1, wm1, wp1, b1b, wys0, wys1, bs, g1c, be1c,
      wc2, wm2, wp2, b2b, g2c, be2c, maskv, gmask)

    return out.reshape(n, d, hgt, wid)


# speedup vs baseline: 1.2841x; 1.0047x over previous
from functools import partial

import numpy as np
import jax
import jax.numpy as jnp
from jax import lax
from jax.experimental import pallas as pl
from jax.experimental.pallas import tpu as pltpu

_EPS = 1e-5
_INV_SQRT2 = 0.7071067811865475


def _gelu_exact(v):
    return 0.5 * v * (1.0 + lax.erf(v * _INV_SQRT2))


def _combine_dx(u, b, pp):
    c = u[1].astype(jnp.bfloat16)
    l = u[2].astype(jnp.bfloat16)
    r = u[0].astype(jnp.bfloat16)
    left = jnp.concatenate([l[:, 1:], l[:, :1]], axis=1)
    right = jnp.concatenate([r[:, -1:], r[:, :-1]], axis=1)
    return c + left + right + b


def _zero_margins(slab, rows, m, pp):
    z = jnp.zeros((rows, m), jnp.bfloat16)
    slab[:, 0:m] = z
    slab[:, m + pp:m + pp + m] = z


def _fill_rows(slab, r0, val, nrows, width, stride, m, row0, rstep):
    for i in range(nrows):
        r = row0 + i * rstep
        slab[r0:r0 + val.shape[0],
             m + i * stride:m + i * stride + width] = (
            val[:, r * width:(r + 1) * width])


def _zero_guards(slab, rows, nrows, width, stride, m):
    zg = jnp.zeros((rows, stride - width), jnp.bfloat16)
    for i in range(nrows):
        slab[:, m + i * stride + width:m + (i + 1) * stride] = zg


def _bn_from_sums(s_ref, q_ref, g_ref, be_ref, count):
    tot = s_ref[...]
    totsq = q_ref[...]
    mu = tot / count
    var = totsq / count - mu * mu
    inv = lax.rsqrt(jnp.maximum(var, 0.0) + _EPS)
    sc = g_ref[...] * inv
    sh = be_ref[...] - mu * sc
    return sc, sh


def _conv9(c_sl, m_sl, p_sl, wc_ref, wm_ref, wp_ref, b, *, pp, d):
    u0 = [None, None, None]
    u1 = [None, None, None]
    for j in range(3):
        t = jnp.dot(wc_ref[j], c_sl, preferred_element_type=jnp.float32)
        u0[j] = t[:d] + jnp.dot(wm_ref[j], m_sl,
                                preferred_element_type=jnp.float32)
        u1[j] = t[d:] + jnp.dot(wp_ref[j], p_sl,
                                preferred_element_type=jnp.float32)
    return _combine_dx(u0, b, pp), _combine_dx(u1, b, pp)


def _fused(x_ref, skip_ref, muph_ref, wc1_ref, wm1_ref, wp1_ref, b1_ref,
           wys0_ref, wys1_ref, bs_ref, g1_ref, be1_ref,
           wc2_ref, wm2_ref, wp2_ref, b2_ref, g2_ref, be2_ref,
           mv_ref, gm_ref,
           out_ref,
           mega_ref,
           y1s_ref, yss_ref,
           s1a_ref, q1a_ref, s2a_ref, q2a_ref,
           sc1_ref, sh1_ref, sc2_ref, sh2_ref,
           *, stride, margin, pp, hh, width, count):
    ph = pl.program_id(0)
    i = pl.program_id(1)
    nimg = x_ref.shape[0]
    c2, p4 = x_ref.shape[1], x_ref.shape[2]
    d = skip_ref.shape[1]
    m = margin
    mv = mv_ref[...]

    @pl.when((ph == 0) & (i == 0))
    def _init_mega():
        _zero_margins(mega_ref, d + c2 + d, m, pp)
        _zero_guards(mega_ref, d + c2 + d, hh, width, stride, m)

    @pl.when(ph == 0)
    def _conv1():
        s = q = None
        for k in range(nimg):
            img = i * nimg + k

            uph = jnp.dot(x_ref[k].astype(jnp.bfloat16),
                          muph_ref[...], preferred_element_type=jnp.float32)
            mega_ref[d:d + c2, m:m + pp] = uph.astype(jnp.bfloat16)

            sk = skip_ref[k].astype(jnp.bfloat16)
            _fill_rows(mega_ref, 0, sk, hh, width, stride, m, 1, 2)
            _fill_rows(mega_ref, d + c2, sk, hh, width, stride, m, 0, 2)

            raw0, raw1 = _conv9(
                mega_ref[:, m:m + pp],
                mega_ref[0:d + c2, m - stride:m - stride + pp],
                mega_ref[d:, m + stride:m + stride + pp],
                wc1_ref, wm1_ref, wp1_ref, b1_ref[...], pp=pp, d=d)

            bsv = bs_ref[...]
            t_up = jnp.dot(wys0_ref[...], mega_ref[d:d + c2, m:m + pp],
                           preferred_element_type=jnp.float32) + bsv
            yss_ref[img, 0] = (t_up + jnp.dot(
                wys1_ref[...], mega_ref[d + c2:, m:m + pp],
                preferred_element_type=jnp.float32)).astype(jnp.bfloat16)
            yss_ref[img, 1] = (t_up + jnp.dot(
                wys1_ref[...], mega_ref[:d, m:m + pp],
                preferred_element_type=jnp.float32)).astype(jnp.bfloat16)

            y1s_ref[img, 0] = raw0
            y1s_ref[img, 1] = raw1
            rc = jnp.concatenate([raw0, raw1], axis=0)
            ts = jnp.dot(rc, mv, preferred_element_type=jnp.float32)
            tq = jnp.dot(rc * rc, mv, preferred_element_type=jnp.float32)
            sk_ = ts[:d] + ts[d:]
            qk = tq[:d] + tq[d:]
            s = sk_ if s is None else s + sk_
            q = qk if q is None else q + qk
        s1a_ref[...] = jnp.where(i == 0, s, s1a_ref[...] + s)
        q1a_ref[...] = jnp.where(i == 0, q, q1a_ref[...] + q)

    @pl.when((ph == 1) & (i == 0))
    def _bn1():
        sc, sh = _bn_from_sums(s1a_ref, q1a_ref, g1_ref, be1_ref, count)
        sc1_ref[...] = sc
        sh1_ref[...] = sh

    @pl.when(ph == 1)
    def _conv2():
        s = q = None
        sc, sh, gm = sc1_ref[...], sh1_ref[...], gm_ref[...]
        for k in range(nimg):
            img = i * nimg + k

            act1 = _gelu_exact(y1s_ref[img, 1].astype(jnp.float32)
                               * sc + sh).astype(jnp.bfloat16) * gm
            act0 = _gelu_exact(y1s_ref[img, 0].astype(jnp.float32)
                               * sc + sh).astype(jnp.bfloat16) * gm

            zr = jnp.zeros((d, stride), jnp.bfloat16)
            acat = jnp.concatenate([act1, act0], axis=0)
            a1m = jnp.concatenate([zr, act1[:, :pp - stride]], axis=1)
            a0p = jnp.concatenate([act0[:, stride:], zr], axis=1)
            raw0, raw1 = _conv9(acat, a1m, a0p, wc2_ref, wm2_ref, wp2_ref,
                                b2_ref[...], pp=pp, d=d)

            y1s_ref[img, 0] = raw0
            y1s_ref[img, 1] = raw1
            rc = jnp.concatenate([raw0, raw1], axis=0)
            ts = jnp.dot(rc, mv, preferred_element_type=jnp.float32)
            tq = jnp.dot(rc * rc, mv, preferred_element_type=jnp.float32)
            sk_ = ts[:d] + ts[d:]
            qk = tq[:d] + tq[d:]
            s = sk_ if s is None else s + sk_
            q = qk if q is None else q + qk
        s2a_ref[...] = jnp.where(i == 0, s, s2a_ref[...] + s)
        q2a_ref[...] = jnp.where(i == 0, q, q2a_ref[...] + q)

    @pl.when((ph == 2) & (i == 0))
    def _bn2():
        sc, sh = _bn_from_sums(s2a_ref, q2a_ref, g2_ref, be2_ref, count)
        sc2_ref[...] = sc
        sh2_ref[...] = sh

    @pl.when(ph == 2)
    def _epilogue():
        sc, sh = sc2_ref[...], sh2_ref[...]
        for k in range(nimg):
            img = i * nimg + k
            for py in range(2):
                act = _gelu_exact(y1s_ref[img, py].astype(jnp.float32)
                                  * sc + sh)
                v = act + yss_ref[img, py].astype(jnp.float32)
                for r in range(hh):
                    fr = 2 * r + py
                    out_ref[k, :, fr * width:(fr + 1) * width] = (
                        v[:, r * stride:r * stride + width])


def kernel(x, skip, w1, b1, g1, be1, w2, b2, g2, be2, wsx, wss, bs):
    n, c2, hh, ww = x.shape
    _, d, hgt, wid = skip.shape
    p4, p = hh * ww, hgt * wid
    stride = wid + 2
    pp = hh * stride
    m = max(128, pl.cdiv(stride + 1, 128) * 128)
    slen = 2 * m + pp
    bf16, f32 = jnp.bfloat16, jnp.float32

    xf = x.reshape(n, c2, p4)
    sf = skip.reshape(n, d, p)

    wu = w1[:, :, :c2]
    wsk = w1[:, :, c2:]

    def taps(w, dy):
        return jnp.stack([w[(dy + 1) * 3 + (dx + 1)] for dx in (-1, 0, 1)])

    wc1 = jnp.concatenate([
        jnp.concatenate([taps(wsk, 1), taps(wu, 0) + taps(wu, 1),
                         taps(wsk, 0)], axis=2),
        jnp.concatenate([taps(wsk, 0), taps(wu, -1) + taps(wu, 0),
                         taps(wsk, -1)], axis=2)], axis=1).astype(bf16)
    wm1 = jnp.concatenate([taps(wsk, -1), taps(wu, -1)], axis=2).astype(bf16)
    wp1 = jnp.concatenate([taps(wu, 1), taps(wsk, 1)], axis=2).astype(bf16)
    wc2 = jnp.concatenate([
        jnp.concatenate([taps(w2, 1), taps(w2, 0)], axis=2),
        jnp.concatenate([taps(w2, 0), taps(w2, -1)], axis=2)],
        axis=1).astype(bf16)
    wm2 = taps(w2, -1).astype(bf16)
    wp2 = taps(w2, 1).astype(bf16)
    wys0 = wsx.astype(bf16)
    wys1 = wss.astype(bf16)

    rr = np.arange(pp) // stride
    cc = np.arange(pp) % stride
    interior = cc < wid
    src = np.where(interior, rr * ww + np.minimum(cc, wid - 1) // 2, -1)
    muph = jnp.asarray(np.arange(p4)[:, None] == src[None, :], bf16)
    maskv = jnp.asarray(interior[:, None], bf16)
    gmask = jnp.asarray(interior[None, :], bf16)

    g1c, be1c = g1.reshape(d, 1), be1.reshape(d, 1)
    g2c, be2c = g2.reshape(d, 1), be2.reshape(d, 1)
    b1b, b2b = b1.astype(bf16), b2.astype(bf16)
    c3s = d + c2 + d

    imgs = next(k for k in (4, 2, 1) if n % k == 0)

    def img0(pidx, ii):
        return (jnp.where(pidx == 0, ii, 0), 0, 0)

    cnst = lambda pidx, ii: (0, 0)
    cnst3 = lambda pidx, ii: (0, 0, 0)

    out = pl.pallas_call(
        partial(_fused, stride=stride, margin=m, pp=pp, hh=hh, width=wid,
                count=float(n * p)),
        grid=(3, n // imgs),
        in_specs=[
            pl.BlockSpec((imgs, c2, p4), img0),
            pl.BlockSpec((imgs, d, p), img0),
            pl.BlockSpec((p4, pp), cnst),
            pl.BlockSpec((3, 2 * d, c3s), cnst3),
            pl.BlockSpec((3, d, d + c2), cnst3),
            pl.BlockSpec((3, d, c2 + d), cnst3),
            pl.BlockSpec((d, 1), cnst),
            pl.BlockSpec((d, c2), cnst),
            pl.BlockSpec((d, d), cnst),
            pl.BlockSpec((d, 1), cnst),
            pl.BlockSpec((d, 1), cnst),
            pl.BlockSpec((d, 1), cnst),
            pl.BlockSpec((3, 2 * d, 2 * d), cnst3),
            pl.BlockSpec((3, d, d), cnst3),
            pl.BlockSpec((3, d, d), cnst3),
            pl.BlockSpec((d, 1), cnst),
            pl.BlockSpec((d, 1), cnst),
            pl.BlockSpec((d, 1), cnst),
            pl.BlockSpec((pp, 1), cnst),
            pl.BlockSpec((1, pp), cnst),
        ],
        out_specs=pl.BlockSpec(
            (imgs, d, p),
            lambda pidx, ii: (jnp.where(pidx == 2, ii, 0), 0, 0)),
        out_shape=jax.ShapeDtypeStruct((n, d, p), f32),
        scratch_shapes=[
            pltpu.VMEM((c3s, slen), bf16),
            pltpu.VMEM((n, 2, d, pp), bf16),
            pltpu.VMEM((n, 2, d, pp), bf16),
            pltpu.VMEM((d, 1), f32),
            pltpu.VMEM((d, 1), f32),
            pltpu.VMEM((d, 1), f32),
            pltpu.VMEM((d, 1), f32),
            pltpu.VMEM((d, 1), f32),
            pltpu.VMEM((d, 1), f32),
            pltpu.VMEM((d, 1), f32),
            pltpu.VMEM((d, 1), f32),
        ],
        compiler_params=pltpu.CompilerParams(
            dimension_semantics=("arbitrary", "arbitrary"),
            vmem_limit_bytes=61_000_000),
    )(xf, sf, muph, wc1, wm1, wp1, b1b, wys0, wys1, bs, g1c, be1c,
      wc2, wm2, wp2, b2b, g2c, be2c, maskv, gmask)

    return out.reshape(n, d, hgt, wid)
